# Initial kernel scaffold; baseline (speedup 1.0000x reference)
import jax
import jax.numpy as jnp
from jax.experimental import pallas as pl


def kernel(tgt, query_pos, reference_points, src, src_spatial_shapes, level_start_index, src_padding_mask, W_so, b_so, W_aw, b_aw, W_vp, b_vp, W_op, b_op, ln1_g, ln1_b, W_fc, b_fc, W_pr, b_pr, ln2_g, ln2_b):
    raise NotImplementedError("write your pallas kernel here")


# speedup vs baseline: 185.3212x; 185.3212x over previous
import functools

import jax
import jax.numpy as jnp
import numpy as np
from jax.experimental import pallas as pl
from jax.experimental.pallas import tpu as pltpu
from jax.experimental.pallas import tpu_sc as plsc

D_MODEL = 256
N_HEAD = 8
N_LEVELS = 4
N_POINTS = 4
HEAD_DIM = 32
B = 4
LQ = 900
LQP = 904
SPATIAL_SHAPES = [(100, 100), (50, 50), (25, 25), (13, 13)]
LEVEL_START = [0, 10000, 12500, 13125]
LEN_IN = 13294
NCOL = 128
N_CORNER = 4
N_IDX = B * LQP * N_CORNER * NCOL

_m_of_col = np.arange(NCOL) // 16
_l_of_col = (np.arange(NCOL) // 4) % 4
_W_of_col = np.array([SPATIAL_SHAPES[l][1] for l in _l_of_col], np.float32)
_H_of_col = np.array([SPATIAL_SHAPES[l][0] for l in _l_of_col], np.float32)
_LS_of_col = np.array([LEVEL_START[l] for l in _l_of_col], np.int32)

WVEC = _W_of_col[None, :]
HVEC = _H_of_col[None, :]
WVEC_I = _W_of_col.astype(np.int32)[None, :]
LSVEC_I = _LS_of_col[None, :]
MVEC_I = _m_of_col.astype(np.int32)[None, :]

_SelX = np.zeros((8, NCOL), np.float32)
_SelY = np.zeros((8, NCOL), np.float32)
for c in range(NCOL):
    l = _l_of_col[c]
    _SelX[2 * l + 0, c] = SPATIAL_SHAPES[l][1]
    _SelY[2 * l + 1, c] = SPATIAL_SHAPES[l][0]

_GM = np.kron(np.eye(8, dtype=np.float32), np.ones((16, 16), np.float32))

_perm = np.empty(256, np.int64)
for c in range(NCOL):
    m, l, p = c // 16, (c // 4) % 4, c % 4
    _perm[c] = ((m * 4 + l) * 4 + p) * 2
    _perm[128 + c] = ((m * 4 + l) * 4 + p) * 2 + 1


def _value_proj_body(src_ref, w_ref, b_ref, mask_ref, out_ref):
    v = jnp.dot(src_ref[0], w_ref[...], preferred_element_type=jnp.float32)
    out_ref[0] = (v + b_ref[...]) * mask_ref[0]


def _value_proj(src, W_vp, b_vp, maskf):
    n_blk = pl.cdiv(LEN_IN, 256)
    return pl.pallas_call(
        _value_proj_body,
        grid=(B, n_blk),
        in_specs=[
            pl.BlockSpec((1, 256, D_MODEL), lambda b, i: (b, i, 0)),
            pl.BlockSpec((D_MODEL, D_MODEL), lambda b, i: (0, 0)),
            pl.BlockSpec((1, D_MODEL), lambda b, i: (0, 0)),
            pl.BlockSpec((1, 256, 1), lambda b, i: (b, i, 0)),
        ],
        out_specs=pl.BlockSpec((1, 256, D_MODEL), lambda b, i: (b, i, 0)),
        out_shape=jax.ShapeDtypeStruct((B, LEN_IN, D_MODEL), jnp.float32),
    )(src, W_vp, b_vp, maskf)


def _qproj_body(tgt_ref, qpos_ref, ref_ref, wso_ref, bso_ref, waw_ref,
                baw_ref, gm_ref, sel_ref, cst_ref, *out_refs):
    b = pl.program_id(0)
    q = tgt_ref[0] + qpos_ref[0]
    so = jnp.dot(q, wso_ref[...], preferred_element_type=jnp.float32) + bso_ref[...]
    logits = jnp.dot(q, waw_ref[...], preferred_element_type=jnp.float32) + baw_ref[...]
    e = jnp.exp(logits - jnp.max(logits, axis=-1, keepdims=True))
    denom = jnp.dot(e, gm_ref[...], preferred_element_type=jnp.float32)
    aw = e / denom

    x = jnp.dot(ref_ref[0], sel_ref[:8],
                preferred_element_type=jnp.float32) + so[:, :NCOL] - 0.5
    y = jnp.dot(ref_ref[0], sel_ref[8:],
                preferred_element_type=jnp.float32) + so[:, NCOL:] - 0.5
    x0 = jnp.floor(x)
    y0 = jnp.floor(y)
    fx = x - x0
    fy = y - y0
    wvec = cst_ref[0:1]
    hvec = cst_ref[1:2]
    wvec_i = wvec.astype(jnp.int32)
    lsm_i = (cst_ref[2:3] * N_HEAD + cst_ref[3:4]).astype(jnp.int32)
    base = b * (LEN_IN * N_HEAD)
    for c, (dx, dy) in enumerate(((0, 0), (1, 0), (0, 1), (1, 1))):
        ix = x0 + dx
        iy = y0 + dy
        vx = ((ix >= 0.0) & (ix <= wvec - 1.0)).astype(jnp.float32)
        vy = ((iy >= 0.0) & (iy <= hvec - 1.0)).astype(jnp.float32)
        ixc = jnp.clip(ix, 0.0, wvec - 1.0).astype(jnp.int32)
        iyc = jnp.clip(iy, 0.0, hvec - 1.0).astype(jnp.int32)
        wx = fx if dx else 1.0 - fx
        wy = fy if dy else 1.0 - fy
        sp = iyc * wvec_i + ixc
        out_refs[c][0] = base + sp * N_HEAD + lsm_i
        out_refs[4 + c][0] = wx * wy * vx * vy * aw


def _qproj(tgt_p, qpos_p, ref_p, W_so_p, b_so_p, W_aw, b_aw):
    qspec = pl.BlockSpec((1, LQP, D_MODEL), lambda b: (b, 0, 0))
    ospec = pl.BlockSpec((1, LQP, NCOL), lambda b: (b, 0, 0))
    oshape = [jax.ShapeDtypeStruct((B, LQP, NCOL), jnp.int32)] * 4 + \
             [jax.ShapeDtypeStruct((B, LQP, NCOL), jnp.float32)] * 4
    gm = jnp.asarray(_GM)
    sel = jnp.asarray(np.concatenate([_SelX, _SelY], axis=0))
    cst = jnp.asarray(np.stack([_W_of_col, _H_of_col,
                                _LS_of_col.astype(np.float32),
                                _m_of_col.astype(np.float32)], axis=0))
    return pl.pallas_call(
        _qproj_body,
        grid=(B,),
        in_specs=[
            qspec, qspec,
            pl.BlockSpec((1, LQP, 8), lambda b: (b, 0, 0)),
            pl.BlockSpec((D_MODEL, 256), lambda b: (0, 0)),
            pl.BlockSpec((1, 256), lambda b: (0, 0)),
            pl.BlockSpec((D_MODEL, NCOL), lambda b: (0, 0)),
            pl.BlockSpec((1, NCOL), lambda b: (0, 0)),
            pl.BlockSpec((NCOL, NCOL), lambda b: (0, 0)),
            pl.BlockSpec((16, NCOL), lambda b: (0, 0)),
            pl.BlockSpec((4, NCOL), lambda b: (0, 0)),
        ],
        out_specs=[ospec] * 8,
        out_shape=oshape,
    )(tgt_p, qpos_p, ref_p, W_so_p, b_so_p, W_aw, b_aw, gm, sel, cst)


_GWIN = 128


def _sc_gather(table, idx_flat):
    mesh = plsc.VectorSubcoreMesh(core_axis_name="c", subcore_axis_name="s")

    @functools.partial(
        pl.kernel,
        out_type=jax.ShapeDtypeStruct((N_IDX, HEAD_DIM), jnp.float32),
        mesh=mesh,
        compiler_params=pltpu.CompilerParams(use_tc_tiling_on_sc=False),
    )
    def k(table_hbm, i_hbm, o_hbm):
        def body(i_vmem, o_vmem):
            pltpu.sync_copy(table_hbm.at[i_vmem.at[0]], o_vmem)

        pltpu.emit_pipeline(
            body,
            grid=(N_IDX // _GWIN,),
            in_specs=[pl.BlockSpec((1, _GWIN), lambda i: (0, i))],
            out_specs=[pl.BlockSpec((_GWIN, HEAD_DIM), lambda i: (i, 0))],
            core_axis_name=("c", "s"),
            dimension_semantics=(pltpu.PARALLEL,),
        )(i_hbm, o_hbm)

    return k(table, idx_flat)


_QB = 32


def _reduce_body(g_ref, w_ref, out_ref):
    p = w_ref[...][..., None] * g_ref[...]
    out_ref[...] = p.sum(axis=3).sum(axis=1)


def _reduce(g5, w5):
    nq = B * LQP
    return pl.pallas_call(
        _reduce_body,
        grid=(nq // _QB,),
        in_specs=[
            pl.BlockSpec((_QB, N_CORNER, N_HEAD, 16, HEAD_DIM),
                         lambda i: (i, 0, 0, 0, 0)),
            pl.BlockSpec((_QB, N_CORNER, N_HEAD, 16), lambda i: (i, 0, 0, 0)),
        ],
        out_specs=pl.BlockSpec((_QB, N_HEAD, HEAD_DIM), lambda i: (i, 0, 0)),
        out_shape=jax.ShapeDtypeStruct((nq, N_HEAD, HEAD_DIM), jnp.float32),
    )(g5, w5)


def _layer_norm(x, g, b):
    m = jnp.mean(x, axis=-1, keepdims=True)
    v = jnp.mean((x - m) ** 2, axis=-1, keepdims=True)
    return (x - m) * jax.lax.rsqrt(v + 1e-5) * g + b


def _post_body(attn_ref, tgt_ref, wop_ref, bop_ref, g1_ref, b1_ref,
               wfc_ref, bfc_ref, wpr_ref, bpr_ref, g2_ref, b2_ref, out_ref):
    tgt2 = jnp.dot(attn_ref[0], wop_ref[...],
                   preferred_element_type=jnp.float32) + bop_ref[...]
    x = _layer_norm(tgt_ref[0] + tgt2, g1_ref[...], b1_ref[...])
    h = jnp.maximum(
        jnp.dot(x, wfc_ref[...], preferred_element_type=jnp.float32)
        + bfc_ref[...], 0.0)
    h = jnp.dot(h, wpr_ref[...], preferred_element_type=jnp.float32) + bpr_ref[...]
    out_ref[0] = _layer_norm(x + h, g2_ref[...], b2_ref[...])


def _post(attn, tgt_p, W_op, b_op, g1, b1, W_fc, b_fc, W_pr, b_pr, g2, b2):
    qspec = pl.BlockSpec((1, LQP, D_MODEL), lambda b: (b, 0, 0))
    return pl.pallas_call(
        _post_body,
        grid=(B,),
        in_specs=[
            qspec, qspec,
            pl.BlockSpec((D_MODEL, D_MODEL), lambda b: (0, 0)),
            pl.BlockSpec((1, D_MODEL), lambda b: (0, 0)),
            pl.BlockSpec((1, D_MODEL), lambda b: (0, 0)),
            pl.BlockSpec((1, D_MODEL), lambda b: (0, 0)),
            pl.BlockSpec((D_MODEL, 4 * D_MODEL), lambda b: (0, 0)),
            pl.BlockSpec((1, 4 * D_MODEL), lambda b: (0, 0)),
            pl.BlockSpec((4 * D_MODEL, D_MODEL), lambda b: (0, 0)),
            pl.BlockSpec((1, D_MODEL), lambda b: (0, 0)),
            pl.BlockSpec((1, D_MODEL), lambda b: (0, 0)),
            pl.BlockSpec((1, D_MODEL), lambda b: (0, 0)),
        ],
        out_specs=qspec,
        out_shape=jax.ShapeDtypeStruct((B, LQP, D_MODEL), jnp.float32),
    )(attn, tgt_p, W_op, b_op, g1, b1, W_fc, b_fc, W_pr, b_pr, g2, b2)


def kernel(tgt, query_pos, reference_points, src, src_spatial_shapes,
           level_start_index, src_padding_mask, W_so, b_so, W_aw, b_aw,
           W_vp, b_vp, W_op, b_op, ln1_g, ln1_b, W_fc, b_fc, W_pr, b_pr,
           ln2_g, ln2_b):
    maskf = (1.0 - src_padding_mask.astype(jnp.float32))[..., None]
    value = _value_proj(src, W_vp, b_vp.reshape(1, -1), maskf)
    table = value.reshape(B * LEN_IN * N_HEAD, HEAD_DIM)

    pad_q = [(0, 0), (0, LQP - LQ), (0, 0)]
    tgt_p = jnp.pad(tgt, pad_q)
    qpos_p = jnp.pad(query_pos, pad_q)
    ref_p = jnp.pad(reference_points.reshape(B, LQ, 8), pad_q)
    W_so_p = W_so[:, jnp.asarray(_perm)]
    b_so_p = b_so[jnp.asarray(_perm)].reshape(1, -1)

    outs = _qproj(tgt_p, qpos_p, ref_p, W_so_p, b_so_p, W_aw,
                  b_aw.reshape(1, -1))
    idx = jnp.stack(outs[:4], axis=2).reshape(1, N_IDX)
    w5 = jnp.stack(outs[4:], axis=2).reshape(B * LQP, N_CORNER, N_HEAD, 16)

    g = _sc_gather(table, idx)
    g5 = g.reshape(B * LQP, N_CORNER, N_HEAD, 16, HEAD_DIM)

    attn = _reduce(g5, w5).reshape(B, LQP, D_MODEL)
    out = _post(attn, tgt_p, W_op, b_op.reshape(1, -1),
                ln1_g.reshape(1, -1), ln1_b.reshape(1, -1),
                W_fc, b_fc.reshape(1, -1), W_pr, b_pr.reshape(1, -1),
                ln2_g.reshape(1, -1), ln2_b.reshape(1, -1))
    return out[:, :LQ, :]


# speedup vs baseline: 303.4641x; 1.6375x over previous
import functools

import jax
import jax.numpy as jnp
import numpy as np
from jax.experimental import pallas as pl
from jax.experimental.pallas import tpu as pltpu
from jax.experimental.pallas import tpu_sc as plsc

D_MODEL = 256
N_HEAD = 8
N_LEVELS = 4
N_POINTS = 4
HEAD_DIM = 32
B = 4
LQ = 900
LQP = 912
SPATIAL_SHAPES = [(100, 100), (50, 50), (25, 25), (13, 13)]
LEVEL_START = [0, 10000, 12500, 13125]
LEN_IN = 13294
NCOL = 128
N_CORNER = 4
N_IDX = B * LQP * N_CORNER * NCOL

_m_of_col = np.arange(NCOL) // 16
_l_of_col = (np.arange(NCOL) // 4) % 4
_W_of_col = np.array([SPATIAL_SHAPES[l][1] for l in _l_of_col], np.float32)
_H_of_col = np.array([SPATIAL_SHAPES[l][0] for l in _l_of_col], np.float32)
_LS_of_col = np.array([LEVEL_START[l] for l in _l_of_col], np.int32)

WVEC = _W_of_col[None, :]
HVEC = _H_of_col[None, :]
WVEC_I = _W_of_col.astype(np.int32)[None, :]
LSVEC_I = _LS_of_col[None, :]
MVEC_I = _m_of_col.astype(np.int32)[None, :]

_SelX = np.zeros((8, NCOL), np.float32)
_SelY = np.zeros((8, NCOL), np.float32)
for c in range(NCOL):
    l = _l_of_col[c]
    _SelX[2 * l + 0, c] = SPATIAL_SHAPES[l][1]
    _SelY[2 * l + 1, c] = SPATIAL_SHAPES[l][0]

_GM = np.kron(np.eye(8, dtype=np.float32), np.ones((16, 16), np.float32))

_perm = np.empty(256, np.int64)
for c in range(NCOL):
    m, l, p = c // 16, (c // 4) % 4, c % 4
    _perm[c] = ((m * 4 + l) * 4 + p) * 2
    _perm[128 + c] = ((m * 4 + l) * 4 + p) * 2 + 1


def _value_proj_body(src_ref, w_ref, b_ref, mask_ref, out_ref):
    v = jnp.dot(src_ref[0], w_ref[...], preferred_element_type=jnp.float32)
    out_ref[0] = (v + b_ref[...]) * mask_ref[0]


def _value_proj(src, W_vp, b_vp, maskf):
    n_blk = pl.cdiv(LEN_IN, 256)
    return pl.pallas_call(
        _value_proj_body,
        grid=(B, n_blk),
        in_specs=[
            pl.BlockSpec((1, 256, D_MODEL), lambda b, i: (b, i, 0)),
            pl.BlockSpec((D_MODEL, D_MODEL), lambda b, i: (0, 0)),
            pl.BlockSpec((1, D_MODEL), lambda b, i: (0, 0)),
            pl.BlockSpec((1, 256, 1), lambda b, i: (b, i, 0)),
        ],
        out_specs=pl.BlockSpec((1, 256, D_MODEL), lambda b, i: (b, i, 0)),
        out_shape=jax.ShapeDtypeStruct((B, LEN_IN, D_MODEL), jnp.float32),
    )(src, W_vp, b_vp, maskf)


def _qproj_body(tgt_ref, qpos_ref, ref_ref, wso_ref, bso_ref, waw_ref,
                baw_ref, gm_ref, sel_ref, cst_ref, *out_refs):
    b = pl.program_id(0)
    q = tgt_ref[0] + qpos_ref[0]
    so = jnp.dot(q, wso_ref[...], preferred_element_type=jnp.float32) + bso_ref[...]
    logits = jnp.dot(q, waw_ref[...], preferred_element_type=jnp.float32) + baw_ref[...]
    e = jnp.exp(logits - jnp.max(logits, axis=-1, keepdims=True))
    denom = jnp.dot(e, gm_ref[...], preferred_element_type=jnp.float32)
    aw = e / denom

    x = jnp.dot(ref_ref[0], sel_ref[:8],
                preferred_element_type=jnp.float32) + so[:, :NCOL] - 0.5
    y = jnp.dot(ref_ref[0], sel_ref[8:],
                preferred_element_type=jnp.float32) + so[:, NCOL:] - 0.5
    x0 = jnp.floor(x)
    y0 = jnp.floor(y)
    fx = x - x0
    fy = y - y0
    wvec = cst_ref[0:1]
    hvec = cst_ref[1:2]
    wvec_i = wvec.astype(jnp.int32)
    lsm_i = (cst_ref[2:3] * N_HEAD + cst_ref[3:4]).astype(jnp.int32)
    base = b * (LEN_IN * N_HEAD)
    for c, (dx, dy) in enumerate(((0, 0), (1, 0), (0, 1), (1, 1))):
        ix = x0 + dx
        iy = y0 + dy
        vx = ((ix >= 0.0) & (ix <= wvec - 1.0)).astype(jnp.float32)
        vy = ((iy >= 0.0) & (iy <= hvec - 1.0)).astype(jnp.float32)
        ixc = jnp.clip(ix, 0.0, wvec - 1.0).astype(jnp.int32)
        iyc = jnp.clip(iy, 0.0, hvec - 1.0).astype(jnp.int32)
        wx = fx if dx else 1.0 - fx
        wy = fy if dy else 1.0 - fy
        sp = iyc * wvec_i + ixc
        out_refs[c][0] = base + sp * N_HEAD + lsm_i
        out_refs[4 + c][0] = wx * wy * vx * vy * aw


def _qproj(tgt_p, qpos_p, ref_p, W_so_p, b_so_p, W_aw, b_aw):
    qspec = pl.BlockSpec((1, LQP, D_MODEL), lambda b: (b, 0, 0))
    ospec = pl.BlockSpec((1, LQP, NCOL), lambda b: (b, 0, 0))
    oshape = [jax.ShapeDtypeStruct((B, LQP, NCOL), jnp.int32)] * 4 + \
             [jax.ShapeDtypeStruct((B, LQP, NCOL), jnp.float32)] * 4
    gm = jnp.asarray(_GM)
    sel = jnp.asarray(np.concatenate([_SelX, _SelY], axis=0))
    cst = jnp.asarray(np.stack([_W_of_col, _H_of_col,
                                _LS_of_col.astype(np.float32),
                                _m_of_col.astype(np.float32)], axis=0))
    return pl.pallas_call(
        _qproj_body,
        grid=(B,),
        in_specs=[
            qspec, qspec,
            pl.BlockSpec((1, LQP, 8), lambda b: (b, 0, 0)),
            pl.BlockSpec((D_MODEL, 256), lambda b: (0, 0)),
            pl.BlockSpec((1, 256), lambda b: (0, 0)),
            pl.BlockSpec((D_MODEL, NCOL), lambda b: (0, 0)),
            pl.BlockSpec((1, NCOL), lambda b: (0, 0)),
            pl.BlockSpec((NCOL, NCOL), lambda b: (0, 0)),
            pl.BlockSpec((16, NCOL), lambda b: (0, 0)),
            pl.BlockSpec((4, NCOL), lambda b: (0, 0)),
        ],
        out_specs=[ospec] * 8,
        out_shape=oshape,
    )(tgt_p, qpos_p, ref_p, W_so_p, b_so_p, W_aw, b_aw, gm, sel, cst)


NW = 32
NQ = B * LQP
QPW = NQ // NW
NT = N_CORNER * NCOL


def _sc_attend(table, comb):
    mesh = plsc.VectorSubcoreMesh(core_axis_name="c", subcore_axis_name="s")

    @functools.partial(
        pl.kernel,
        out_type=jax.ShapeDtypeStruct((NQ * N_HEAD, HEAD_DIM), jnp.float32),
        mesh=mesh,
        compiler_params=pltpu.CompilerParams(use_tc_tiling_on_sc=False,
                                             needs_layout_passes=False),
        scratch_types=[
            pltpu.VMEM((1024,), jnp.int32),
            pltpu.VMEM((NT, HEAD_DIM), jnp.float32),
            pltpu.VMEM((N_HEAD, HEAD_DIM), jnp.float32),
            pltpu.SemaphoreType.DMA,
        ],
    )
    def k(table_hbm, comb_hbm, out_hbm, cbuf, gbuf, obuf, gsem):
        iota = jax.lax.iota(jnp.int32, 16)
        wid = jax.lax.axis_index("s") * 2 + jax.lax.axis_index("c")
        base = wid * QPW

        @pl.loop(0, QPW)
        def _(qi):
            qabs = base + qi
            pltpu.sync_copy(comb_hbm.at[qabs], cbuf)
            for c in range(N_CORNER):
                sl = pl.ds(c * 128, 128)
                pltpu.make_async_copy(table_hbm.at[cbuf.at[sl]],
                                      gbuf.at[sl], gsem).start()
            for c in range(N_CORNER):
                sl = pl.ds(c * 128, 128)
                pltpu.make_async_copy(table_hbm.at[cbuf.at[sl]],
                                      gbuf.at[sl], gsem).wait()

            @pl.loop(0, N_HEAD)
            def _(m):
                acc0 = jnp.zeros((16,), jnp.float32)
                acc1 = jnp.zeros((16,), jnp.float32)
                for c in range(N_CORNER):
                    for t in range(16):
                        off = c * 128 + m * 16 + t
                        wi = plsc.load_gather(
                            cbuf, [jnp.full((16,), 512 + off, jnp.int32)])
                        wf = plsc.bitcast(wi, jnp.float32)
                        rows = jnp.full((16,), off, jnp.int32)
                        g0 = plsc.load_gather(gbuf, [rows, iota])
                        g1 = plsc.load_gather(gbuf, [rows, iota + 16])
                        acc0 = acc0 + wf * g0
                        acc1 = acc1 + wf * g1
                mrow = jnp.full((16,), m, jnp.int32)
                plsc.store_scatter(obuf, [mrow, iota], acc0)
                plsc.store_scatter(obuf, [mrow, iota + 16], acc1)

            pltpu.sync_copy(obuf, out_hbm.at[pl.ds(qabs * N_HEAD, N_HEAD)])

    return k(table, comb)


def _layer_norm(x, g, b):
    m = jnp.mean(x, axis=-1, keepdims=True)
    v = jnp.mean((x - m) ** 2, axis=-1, keepdims=True)
    return (x - m) * jax.lax.rsqrt(v + 1e-5) * g + b


def _post_body(attn_ref, tgt_ref, wop_ref, bop_ref, g1_ref, b1_ref,
               wfc_ref, bfc_ref, wpr_ref, bpr_ref, g2_ref, b2_ref, out_ref):
    tgt2 = jnp.dot(attn_ref[0], wop_ref[...],
                   preferred_element_type=jnp.float32) + bop_ref[...]
    x = _layer_norm(tgt_ref[0] + tgt2, g1_ref[...], b1_ref[...])
    h = jnp.maximum(
        jnp.dot(x, wfc_ref[...], preferred_element_type=jnp.float32)
        + bfc_ref[...], 0.0)
    h = jnp.dot(h, wpr_ref[...], preferred_element_type=jnp.float32) + bpr_ref[...]
    out_ref[0] = _layer_norm(x + h, g2_ref[...], b2_ref[...])


def _post(attn, tgt_p, W_op, b_op, g1, b1, W_fc, b_fc, W_pr, b_pr, g2, b2):
    qspec = pl.BlockSpec((1, LQP, D_MODEL), lambda b: (b, 0, 0))
    return pl.pallas_call(
        _post_body,
        grid=(B,),
        in_specs=[
            qspec, qspec,
            pl.BlockSpec((D_MODEL, D_MODEL), lambda b: (0, 0)),
            pl.BlockSpec((1, D_MODEL), lambda b: (0, 0)),
            pl.BlockSpec((1, D_MODEL), lambda b: (0, 0)),
            pl.BlockSpec((1, D_MODEL), lambda b: (0, 0)),
            pl.BlockSpec((D_MODEL, 4 * D_MODEL), lambda b: (0, 0)),
            pl.BlockSpec((1, 4 * D_MODEL), lambda b: (0, 0)),
            pl.BlockSpec((4 * D_MODEL, D_MODEL), lambda b: (0, 0)),
            pl.BlockSpec((1, D_MODEL), lambda b: (0, 0)),
            pl.BlockSpec((1, D_MODEL), lambda b: (0, 0)),
            pl.BlockSpec((1, D_MODEL), lambda b: (0, 0)),
        ],
        out_specs=qspec,
        out_shape=jax.ShapeDtypeStruct((B, LQP, D_MODEL), jnp.float32),
    )(attn, tgt_p, W_op, b_op, g1, b1, W_fc, b_fc, W_pr, b_pr, g2, b2)


def kernel(tgt, query_pos, reference_points, src, src_spatial_shapes,
           level_start_index, src_padding_mask, W_so, b_so, W_aw, b_aw,
           W_vp, b_vp, W_op, b_op, ln1_g, ln1_b, W_fc, b_fc, W_pr, b_pr,
           ln2_g, ln2_b):
    maskf = (1.0 - src_padding_mask.astype(jnp.float32))[..., None]
    value = _value_proj(src, W_vp, b_vp.reshape(1, -1), maskf)
    table = value.reshape(B * LEN_IN * N_HEAD, HEAD_DIM)

    pad_q = [(0, 0), (0, LQP - LQ), (0, 0)]
    tgt_p = jnp.pad(tgt, pad_q)
    qpos_p = jnp.pad(query_pos, pad_q)
    ref_p = jnp.pad(reference_points.reshape(B, LQ, 8), pad_q)
    W_so_p = W_so[:, jnp.asarray(_perm)]
    b_so_p = b_so[jnp.asarray(_perm)].reshape(1, -1)

    outs = _qproj(tgt_p, qpos_p, ref_p, W_so_p, b_so_p, W_aw,
                  b_aw.reshape(1, -1))
    idx = jnp.stack(outs[:4], axis=2).reshape(NQ, NT)
    wbits = jax.lax.bitcast_convert_type(
        jnp.stack(outs[4:], axis=2), jnp.int32).reshape(NQ, NT)
    comb = jnp.concatenate([idx, wbits], axis=1)

    attn = _sc_attend(table, comb).reshape(B, LQP, D_MODEL)
    out = _post(attn, tgt_p, W_op, b_op.reshape(1, -1),
                ln1_g.reshape(1, -1), ln1_b.reshape(1, -1),
                W_fc, b_fc.reshape(1, -1), W_pr, b_pr.reshape(1, -1),
                ln2_g.reshape(1, -1), ln2_b.reshape(1, -1))
    return out[:, :LQ, :]


# speedup vs baseline: 303.5319x; 1.0002x over previous
import functools

import jax
import jax.numpy as jnp
import numpy as np
from jax.experimental import pallas as pl
from jax.experimental.pallas import tpu as pltpu
from jax.experimental.pallas import tpu_sc as plsc

D_MODEL = 256
N_HEAD = 8
N_LEVELS = 4
N_POINTS = 4
HEAD_DIM = 32
B = 4
LQ = 900
LQP = 928
SPATIAL_SHAPES = [(100, 100), (50, 50), (25, 25), (13, 13)]
LEVEL_START = [0, 10000, 12500, 13125]
LEN_IN = 13294
NCOL = 128
N_CORNER = 4
N_IDX = B * LQP * N_CORNER * NCOL

_m_of_col = np.arange(NCOL) // 16
_l_of_col = (np.arange(NCOL) // 4) % 4
_W_of_col = np.array([SPATIAL_SHAPES[l][1] for l in _l_of_col], np.float32)
_H_of_col = np.array([SPATIAL_SHAPES[l][0] for l in _l_of_col], np.float32)
_LS_of_col = np.array([LEVEL_START[l] for l in _l_of_col], np.int32)

WVEC = _W_of_col[None, :]
HVEC = _H_of_col[None, :]
WVEC_I = _W_of_col.astype(np.int32)[None, :]
LSVEC_I = _LS_of_col[None, :]
MVEC_I = _m_of_col.astype(np.int32)[None, :]

_SelX = np.zeros((8, NCOL), np.float32)
_SelY = np.zeros((8, NCOL), np.float32)
for c in range(NCOL):
    l = _l_of_col[c]
    _SelX[2 * l + 0, c] = SPATIAL_SHAPES[l][1]
    _SelY[2 * l + 1, c] = SPATIAL_SHAPES[l][0]

_GM = np.kron(np.eye(8, dtype=np.float32), np.ones((16, 16), np.float32))

_perm = np.empty(256, np.int64)
for c in range(NCOL):
    m, l, p = c // 16, (c // 4) % 4, c % 4
    _perm[c] = ((m * 4 + l) * 4 + p) * 2
    _perm[128 + c] = ((m * 4 + l) * 4 + p) * 2 + 1


def _value_proj_body(src_ref, w_ref, b_ref, mask_ref, out_ref):
    v = jnp.dot(src_ref[0], w_ref[...], preferred_element_type=jnp.float32)
    out_ref[0] = (v + b_ref[...]) * mask_ref[0]


def _value_proj(src, W_vp, b_vp, maskf):
    n_blk = pl.cdiv(LEN_IN, 256)
    return pl.pallas_call(
        _value_proj_body,
        grid=(B, n_blk),
        in_specs=[
            pl.BlockSpec((1, 256, D_MODEL), lambda b, i: (b, i, 0)),
            pl.BlockSpec((D_MODEL, D_MODEL), lambda b, i: (0, 0)),
            pl.BlockSpec((1, D_MODEL), lambda b, i: (0, 0)),
            pl.BlockSpec((1, 256, 1), lambda b, i: (b, i, 0)),
        ],
        out_specs=pl.BlockSpec((1, 256, D_MODEL), lambda b, i: (b, i, 0)),
        out_shape=jax.ShapeDtypeStruct((B, LEN_IN, D_MODEL), jnp.float32),
    )(src, W_vp, b_vp, maskf)


def _qproj_body(tgt_ref, qpos_ref, ref_ref, wso_ref, bso_ref, waw_ref,
                baw_ref, gm_ref, sel_ref, cst_ref, *out_refs):
    b = pl.program_id(0)
    q = tgt_ref[0] + qpos_ref[0]
    so = jnp.dot(q, wso_ref[...], preferred_element_type=jnp.float32) + bso_ref[...]
    logits = jnp.dot(q, waw_ref[...], preferred_element_type=jnp.float32) + baw_ref[...]
    e = jnp.exp(logits - jnp.max(logits, axis=-1, keepdims=True))
    denom = jnp.dot(e, gm_ref[...], preferred_element_type=jnp.float32)
    aw = e / denom

    x = jnp.dot(ref_ref[0], sel_ref[:8],
                preferred_element_type=jnp.float32) + so[:, :NCOL] - 0.5
    y = jnp.dot(ref_ref[0], sel_ref[8:],
                preferred_element_type=jnp.float32) + so[:, NCOL:] - 0.5
    x0 = jnp.floor(x)
    y0 = jnp.floor(y)
    fx = x - x0
    fy = y - y0
    wvec = cst_ref[0:1]
    hvec = cst_ref[1:2]
    wvec_i = wvec.astype(jnp.int32)
    lsm_i = (cst_ref[2:3] * N_HEAD + cst_ref[3:4]).astype(jnp.int32)
    base = b * (LEN_IN * N_HEAD)
    for c, (dx, dy) in enumerate(((0, 0), (1, 0), (0, 1), (1, 1))):
        ix = x0 + dx
        iy = y0 + dy
        vx = ((ix >= 0.0) & (ix <= wvec - 1.0)).astype(jnp.float32)
        vy = ((iy >= 0.0) & (iy <= hvec - 1.0)).astype(jnp.float32)
        ixc = jnp.clip(ix, 0.0, wvec - 1.0).astype(jnp.int32)
        iyc = jnp.clip(iy, 0.0, hvec - 1.0).astype(jnp.int32)
        wx = fx if dx else 1.0 - fx
        wy = fy if dy else 1.0 - fy
        sp = iyc * wvec_i + ixc
        out_refs[c][0] = base + sp * N_HEAD + lsm_i
        out_refs[4 + c][0] = wx * wy * vx * vy * aw


def _qproj(tgt_p, qpos_p, ref_p, W_so_p, b_so_p, W_aw, b_aw):
    qspec = pl.BlockSpec((1, LQP, D_MODEL), lambda b: (b, 0, 0))
    ospec = pl.BlockSpec((1, LQP, NCOL), lambda b: (b, 0, 0))
    oshape = [jax.ShapeDtypeStruct((B, LQP, NCOL), jnp.int32)] * 4 + \
             [jax.ShapeDtypeStruct((B, LQP, NCOL), jnp.float32)] * 4
    gm = jnp.asarray(_GM)
    sel = jnp.asarray(np.concatenate([_SelX, _SelY], axis=0))
    cst = jnp.asarray(np.stack([_W_of_col, _H_of_col,
                                _LS_of_col.astype(np.float32),
                                _m_of_col.astype(np.float32)], axis=0))
    return pl.pallas_call(
        _qproj_body,
        grid=(B,),
        in_specs=[
            qspec, qspec,
            pl.BlockSpec((1, LQP, 8), lambda b: (b, 0, 0)),
            pl.BlockSpec((D_MODEL, 256), lambda b: (0, 0)),
            pl.BlockSpec((1, 256), lambda b: (0, 0)),
            pl.BlockSpec((D_MODEL, NCOL), lambda b: (0, 0)),
            pl.BlockSpec((1, NCOL), lambda b: (0, 0)),
            pl.BlockSpec((NCOL, NCOL), lambda b: (0, 0)),
            pl.BlockSpec((16, NCOL), lambda b: (0, 0)),
            pl.BlockSpec((4, NCOL), lambda b: (0, 0)),
        ],
        out_specs=[ospec] * 8,
        out_shape=oshape,
    )(tgt_p, qpos_p, ref_p, W_so_p, b_so_p, W_aw, b_aw, gm, sel, cst)


NW = 32
NQ = B * LQP
QPW = NQ // NW
NT = N_CORNER * NCOL


def _sc_attend(table, comb):
    mesh = plsc.VectorSubcoreMesh(core_axis_name="c", subcore_axis_name="s")

    nbuf = 4

    @functools.partial(
        pl.kernel,
        out_type=jax.ShapeDtypeStruct((NQ * N_HEAD, HEAD_DIM), jnp.float32),
        mesh=mesh,
        compiler_params=pltpu.CompilerParams(use_tc_tiling_on_sc=False,
                                             needs_layout_passes=False),
        scratch_types=(
            [pltpu.VMEM((1024,), jnp.int32)] * nbuf
            + [pltpu.VMEM((NT, HEAD_DIM), jnp.float32)] * nbuf
            + [pltpu.VMEM((N_HEAD, HEAD_DIM), jnp.float32)] * nbuf
            + [pltpu.SemaphoreType.DMA] * (3 * nbuf)
        ),
    )
    def k(table_hbm, comb_hbm, out_hbm, *scr):
        cbs, gbs, obs = scr[0:4], scr[4:8], scr[8:12]
        css, gss, oss = scr[12:16], scr[16:20], scr[20:24]
        iota = jax.lax.iota(jnp.int32, 16)
        wid = jax.lax.axis_index("s") * 2 + jax.lax.axis_index("c")
        base = wid * QPW

        def startc(ql, j):
            pltpu.make_async_copy(comb_hbm.at[base + ql], cbs[j],
                                  css[j]).start()

        def g_issue(ql, j):
            pltpu.make_async_copy(comb_hbm.at[base + ql], cbs[j],
                                  css[j]).wait()
            for c in range(N_CORNER):
                sl = pl.ds(c * 128, 128)
                pltpu.make_async_copy(table_hbm.at[cbs[j].at[sl]],
                                      gbs[j].at[sl], gss[j]).start()

        def compute(ql, j):
            for c in range(N_CORNER):
                sl = pl.ds(c * 128, 128)
                pltpu.make_async_copy(table_hbm.at[cbs[j].at[sl]],
                                      gbs[j].at[sl], gss[j]).wait()

            @pl.when(ql >= nbuf)
            def _():
                pltpu.make_async_copy(
                    obs[j], out_hbm.at[pl.ds((base + ql) * N_HEAD, N_HEAD)],
                    oss[j]).wait()

            @pl.loop(0, N_HEAD)
            def _(m):
                acc0 = jnp.zeros((16,), jnp.float32)
                acc1 = jnp.zeros((16,), jnp.float32)
                for c in range(N_CORNER):
                    for t in range(16):
                        off = c * 128 + m * 16 + t
                        wi = plsc.load_gather(
                            cbs[j], [jnp.full((16,), 512 + off, jnp.int32)])
                        wf = plsc.bitcast(wi, jnp.float32)
                        rows = jnp.full((16,), off, jnp.int32)
                        g0 = plsc.load_gather(gbs[j], [rows, iota])
                        g1 = plsc.load_gather(gbs[j], [rows, iota + 16])
                        acc0 = acc0 + wf * g0
                        acc1 = acc1 + wf * g1
                mrow = jnp.full((16,), m, jnp.int32)
                plsc.store_scatter(obs[j], [mrow, iota], acc0)
                plsc.store_scatter(obs[j], [mrow, iota + 16], acc1)

            pltpu.make_async_copy(
                obs[j], out_hbm.at[pl.ds((base + ql) * N_HEAD, N_HEAD)],
                oss[j]).start()

        for j in range(nbuf):
            startc(j, j)
        g_issue(0, 0)
        g_issue(1, 1)

        @pl.loop(0, QPW // nbuf)
        def _(kk):
            q0 = kk * nbuf
            for j in range(nbuf):
                ql = q0 + j
                compute(ql, j)

                @pl.when(ql + 2 < QPW)
                def _():
                    g_issue(ql + 2, (j + 2) % nbuf)

                @pl.when(ql + 4 < QPW)
                def _():
                    startc(ql + 4, j)

        for ql in range(QPW - nbuf, QPW):
            j = ql % nbuf
            pltpu.make_async_copy(
                obs[j], out_hbm.at[pl.ds((base + ql) * N_HEAD, N_HEAD)],
                oss[j]).wait()

    return k(table, comb)


def _layer_norm(x, g, b):
    m = jnp.mean(x, axis=-1, keepdims=True)
    v = jnp.mean((x - m) ** 2, axis=-1, keepdims=True)
    return (x - m) * jax.lax.rsqrt(v + 1e-5) * g + b


def _post_body(attn_ref, tgt_ref, wop_ref, bop_ref, g1_ref, b1_ref,
               wfc_ref, bfc_ref, wpr_ref, bpr_ref, g2_ref, b2_ref, out_ref):
    tgt2 = jnp.dot(attn_ref[0], wop_ref[...],
                   preferred_element_type=jnp.float32) + bop_ref[...]
    x = _layer_norm(tgt_ref[0] + tgt2, g1_ref[...], b1_ref[...])
    h = jnp.maximum(
        jnp.dot(x, wfc_ref[...], preferred_element_type=jnp.float32)
        + bfc_ref[...], 0.0)
    h = jnp.dot(h, wpr_ref[...], preferred_element_type=jnp.float32) + bpr_ref[...]
    out_ref[0] = _layer_norm(x + h, g2_ref[...], b2_ref[...])


def _post(attn, tgt_p, W_op, b_op, g1, b1, W_fc, b_fc, W_pr, b_pr, g2, b2):
    qspec = pl.BlockSpec((1, LQP, D_MODEL), lambda b: (b, 0, 0))
    return pl.pallas_call(
        _post_body,
        grid=(B,),
        in_specs=[
            qspec, qspec,
            pl.BlockSpec((D_MODEL, D_MODEL), lambda b: (0, 0)),
            pl.BlockSpec((1, D_MODEL), lambda b: (0, 0)),
            pl.BlockSpec((1, D_MODEL), lambda b: (0, 0)),
            pl.BlockSpec((1, D_MODEL), lambda b: (0, 0)),
            pl.BlockSpec((D_MODEL, 4 * D_MODEL), lambda b: (0, 0)),
            pl.BlockSpec((1, 4 * D_MODEL), lambda b: (0, 0)),
            pl.BlockSpec((4 * D_MODEL, D_MODEL), lambda b: (0, 0)),
            pl.BlockSpec((1, D_MODEL), lambda b: (0, 0)),
            pl.BlockSpec((1, D_MODEL), lambda b: (0, 0)),
            pl.BlockSpec((1, D_MODEL), lambda b: (0, 0)),
        ],
        out_specs=qspec,
        out_shape=jax.ShapeDtypeStruct((B, LQP, D_MODEL), jnp.float32),
    )(attn, tgt_p, W_op, b_op, g1, b1, W_fc, b_fc, W_pr, b_pr, g2, b2)


def kernel(tgt, query_pos, reference_points, src, src_spatial_shapes,
           level_start_index, src_padding_mask, W_so, b_so, W_aw, b_aw,
           W_vp, b_vp, W_op, b_op, ln1_g, ln1_b, W_fc, b_fc, W_pr, b_pr,
           ln2_g, ln2_b):
    maskf = (1.0 - src_padding_mask.astype(jnp.float32))[..., None]
    value = _value_proj(src, W_vp, b_vp.reshape(1, -1), maskf)
    table = value.reshape(B * LEN_IN * N_HEAD, HEAD_DIM)

    pad_q = [(0, 0), (0, LQP - LQ), (0, 0)]
    tgt_p = jnp.pad(tgt, pad_q)
    qpos_p = jnp.pad(query_pos, pad_q)
    ref_p = jnp.pad(reference_points.reshape(B, LQ, 8), pad_q)
    W_so_p = W_so[:, jnp.asarray(_perm)]
    b_so_p = b_so[jnp.asarray(_perm)].reshape(1, -1)

    outs = _qproj(tgt_p, qpos_p, ref_p, W_so_p, b_so_p, W_aw,
                  b_aw.reshape(1, -1))
    idx = jnp.stack(outs[:4], axis=2).reshape(NQ, NT)
    wbits = jax.lax.bitcast_convert_type(
        jnp.stack(outs[4:], axis=2), jnp.int32).reshape(NQ, NT)
    comb = jnp.concatenate([idx, wbits], axis=1)

    attn = _sc_attend(table, comb).reshape(B, LQP, D_MODEL)
    out = _post(attn, tgt_p, W_op, b_op.reshape(1, -1),
                ln1_g.reshape(1, -1), ln1_b.reshape(1, -1),
                W_fc, b_fc.reshape(1, -1), W_pr, b_pr.reshape(1, -1),
                ln2_g.reshape(1, -1), ln2_b.reshape(1, -1))
    return out[:, :LQ, :]


# speedup vs baseline: 628.2434x; 2.0698x over previous
import functools

import jax
import jax.numpy as jnp
import numpy as np
from jax.experimental import pallas as pl
from jax.experimental.pallas import tpu as pltpu
from jax.experimental.pallas import tpu_sc as plsc

D_MODEL = 256
N_HEAD = 8
N_LEVELS = 4
N_POINTS = 4
HEAD_DIM = 32
B = 4
LQ = 900
LQP = 928
SPATIAL_SHAPES = [(100, 100), (50, 50), (25, 25), (13, 13)]
LEVEL_START = [0, 10000, 12500, 13125]
LEN_IN = 13294
LEN_PAD = 13312
NCOL = 128
N_CORNER = 4
N_IDX = B * LQP * N_CORNER * NCOL

_m_of_col = np.arange(NCOL) // 16
_l_of_col = (np.arange(NCOL) // 4) % 4
_W_of_col = np.array([SPATIAL_SHAPES[l][1] for l in _l_of_col], np.float32)
_H_of_col = np.array([SPATIAL_SHAPES[l][0] for l in _l_of_col], np.float32)
_LS_of_col = np.array([LEVEL_START[l] for l in _l_of_col], np.int32)

WVEC = _W_of_col[None, :]
HVEC = _H_of_col[None, :]
WVEC_I = _W_of_col.astype(np.int32)[None, :]
LSVEC_I = _LS_of_col[None, :]
MVEC_I = _m_of_col.astype(np.int32)[None, :]

_SelX = np.zeros((8, NCOL), np.float32)
_SelY = np.zeros((8, NCOL), np.float32)
for c in range(NCOL):
    l = _l_of_col[c]
    _SelX[2 * l + 0, c] = SPATIAL_SHAPES[l][1]
    _SelY[2 * l + 1, c] = SPATIAL_SHAPES[l][0]

_GM = np.kron(np.eye(8, dtype=np.float32), np.ones((16, 16), np.float32))

_perm = np.empty(256, np.int64)
for c in range(NCOL):
    m, l, p = c // 16, (c // 4) % 4, c % 4
    _perm[c] = ((m * 4 + l) * 4 + p) * 2
    _perm[128 + c] = ((m * 4 + l) * 4 + p) * 2 + 1


def _value_proj_body(src_ref, w_ref, b_ref, mask_ref, out_ref):
    v = jnp.dot(src_ref[0], w_ref[...], preferred_element_type=jnp.float32)
    out_ref[...] = ((v + b_ref[...]) * mask_ref[0]).reshape(512, 128)


def _value_proj(src, W_vp, b_vp, maskf):
    n_blk = LEN_PAD // 256
    return pl.pallas_call(
        _value_proj_body,
        grid=(B, n_blk),
        in_specs=[
            pl.BlockSpec((1, 256, D_MODEL), lambda b, i: (b, i, 0)),
            pl.BlockSpec((D_MODEL, D_MODEL), lambda b, i: (0, 0)),
            pl.BlockSpec((1, D_MODEL), lambda b, i: (0, 0)),
            pl.BlockSpec((1, 256, 1), lambda b, i: (b, i, 0)),
        ],
        out_specs=pl.BlockSpec((512, 128), lambda b, i: (b * n_blk + i, 0)),
        out_shape=jax.ShapeDtypeStruct((B * 2 * LEN_PAD, 128), jnp.float32),
    )(src, W_vp, b_vp, maskf)


def _qproj_body(tgt_ref, qpos_ref, ref_ref, wso_ref, bso_ref, waw_ref,
                baw_ref, gm_ref, sel_ref, cst_ref, *out_refs):
    b = pl.program_id(0)
    q = tgt_ref[0] + qpos_ref[0]
    so = jnp.dot(q, wso_ref[...], preferred_element_type=jnp.float32) + bso_ref[...]
    logits = jnp.dot(q, waw_ref[...], preferred_element_type=jnp.float32) + baw_ref[...]
    e = jnp.exp(logits - jnp.max(logits, axis=-1, keepdims=True))
    denom = jnp.dot(e, gm_ref[...], preferred_element_type=jnp.float32)
    aw = e / denom

    x = jnp.dot(ref_ref[0], sel_ref[:8],
                preferred_element_type=jnp.float32) + so[:, :NCOL] - 0.5
    y = jnp.dot(ref_ref[0], sel_ref[8:],
                preferred_element_type=jnp.float32) + so[:, NCOL:] - 0.5
    x0 = jnp.floor(x)
    y0 = jnp.floor(y)
    fx = x - x0
    fy = y - y0
    wvec = cst_ref[0:1]
    hvec = cst_ref[1:2]
    wvec_i = wvec.astype(jnp.int32)
    lsm_i = (cst_ref[2:3] * N_HEAD + cst_ref[3:4]).astype(jnp.int32)
    base = b * (LEN_PAD * N_HEAD)
    for c, (dx, dy) in enumerate(((0, 0), (1, 0), (0, 1), (1, 1))):
        ix = x0 + dx
        iy = y0 + dy
        vx = ((ix >= 0.0) & (ix <= wvec - 1.0)).astype(jnp.float32)
        vy = ((iy >= 0.0) & (iy <= hvec - 1.0)).astype(jnp.float32)
        ixc = jnp.clip(ix, 0.0, wvec - 1.0).astype(jnp.int32)
        iyc = jnp.clip(iy, 0.0, hvec - 1.0).astype(jnp.int32)
        wx = fx if dx else 1.0 - fx
        wy = fy if dy else 1.0 - fy
        sp = iyc * wvec_i + ixc
        out_refs[c][0] = base + sp * N_HEAD + lsm_i
        out_refs[4 + c][0] = wx * wy * vx * vy * aw


def _qproj(tgt_p, qpos_p, ref_p, W_so_p, b_so_p, W_aw, b_aw):
    qspec = pl.BlockSpec((1, LQP, D_MODEL), lambda b: (b, 0, 0))
    ospec = pl.BlockSpec((1, LQP, NCOL), lambda b: (b, 0, 0))
    oshape = [jax.ShapeDtypeStruct((B, LQP, NCOL), jnp.int32)] * 4 + \
             [jax.ShapeDtypeStruct((B, LQP, NCOL), jnp.float32)] * 4
    gm = jnp.asarray(_GM)
    sel = jnp.asarray(np.concatenate([_SelX, _SelY], axis=0))
    cst = jnp.asarray(np.stack([_W_of_col, _H_of_col,
                                _LS_of_col.astype(np.float32),
                                _m_of_col.astype(np.float32)], axis=0))
    return pl.pallas_call(
        _qproj_body,
        grid=(B,),
        in_specs=[
            qspec, qspec,
            pl.BlockSpec((1, LQP, 8), lambda b: (b, 0, 0)),
            pl.BlockSpec((D_MODEL, 256), lambda b: (0, 0)),
            pl.BlockSpec((1, 256), lambda b: (0, 0)),
            pl.BlockSpec((D_MODEL, NCOL), lambda b: (0, 0)),
            pl.BlockSpec((1, NCOL), lambda b: (0, 0)),
            pl.BlockSpec((NCOL, NCOL), lambda b: (0, 0)),
            pl.BlockSpec((16, NCOL), lambda b: (0, 0)),
            pl.BlockSpec((4, NCOL), lambda b: (0, 0)),
        ],
        out_specs=[ospec] * 8,
        out_shape=oshape,
    )(tgt_p, qpos_p, ref_p, W_so_p, b_so_p, W_aw, b_aw, gm, sel, cst)


NW = 32
NQ = B * LQP
QPW = NQ // NW
NT = N_CORNER * NCOL


def _sc_attend(table, comb):
    mesh = plsc.VectorSubcoreMesh(core_axis_name="c", subcore_axis_name="s")

    nbuf = 4

    @functools.partial(
        pl.kernel,
        out_type=jax.ShapeDtypeStruct((NQ * 2, 128), jnp.float32),
        mesh=mesh,
        compiler_params=pltpu.CompilerParams(use_tc_tiling_on_sc=False,
                                             needs_layout_passes=False),
        scratch_types=(
            [pltpu.VMEM((8, 128), jnp.int32)] * nbuf
            + [pltpu.VMEM((NT, HEAD_DIM), jnp.float32)] * nbuf
            + [pltpu.VMEM((2, 128), jnp.float32)] * nbuf
            + [pltpu.SemaphoreType.DMA] * (3 * nbuf)
        ),
    )
    def k(table_hbm, comb_hbm, out_hbm, *scr):
        cbs, gbs, obs = scr[0:4], scr[4:8], scr[8:12]
        css, gss, oss = scr[12:16], scr[16:20], scr[20:24]
        iota = jax.lax.iota(jnp.int32, 16)
        wid = jax.lax.axis_index("s") * 2 + jax.lax.axis_index("c")
        base = wid * QPW

        def startc(ql, j):
            pltpu.make_async_copy(comb_hbm.at[base + ql], cbs[j],
                                  css[j]).start()

        def g_issue(ql, j):
            pltpu.make_async_copy(comb_hbm.at[base + ql], cbs[j],
                                  css[j]).wait()
            for c in range(N_CORNER):
                pltpu.make_async_copy(table_hbm.at[cbs[j].at[c]],
                                      gbs[j].at[pl.ds(c * 128, 128)],
                                      gss[j]).start()

        def compute(ql, j):
            for c in range(N_CORNER):
                pltpu.make_async_copy(table_hbm.at[cbs[j].at[c]],
                                      gbs[j].at[pl.ds(c * 128, 128)],
                                      gss[j]).wait()

            @pl.when(ql >= nbuf)
            def _():
                pltpu.make_async_copy(
                    obs[j], out_hbm.at[pl.ds((base + ql) * 2, 2)],
                    oss[j]).wait()

            @pl.loop(0, N_HEAD)
            def _(m):
                acc0 = jnp.zeros((16,), jnp.float32)
                acc1 = jnp.zeros((16,), jnp.float32)
                for c in range(N_CORNER):
                    for t in range(16):
                        off2 = m * 16 + t
                        wi = plsc.load_gather(
                            cbs[j], [jnp.full((16,), 4 + c, jnp.int32),
                                     jnp.full((16,), off2, jnp.int32)])
                        wf = plsc.bitcast(wi, jnp.float32)
                        rows = jnp.full((16,), c * 128 + off2, jnp.int32)
                        g0 = plsc.load_gather(gbs[j], [rows, iota])
                        g1 = plsc.load_gather(gbs[j], [rows, iota + 16])
                        acc0 = acc0 + wf * g0
                        acc1 = acc1 + wf * g1
                orow0 = jnp.full((16,), m // 4, jnp.int32)
                ocol = (m % 4) * 32 + iota
                plsc.store_scatter(obs[j], [orow0, ocol], acc0)
                plsc.store_scatter(obs[j], [orow0, ocol + 16], acc1)

            pltpu.make_async_copy(
                obs[j], out_hbm.at[pl.ds((base + ql) * 2, 2)],
                oss[j]).start()

        for j in range(nbuf):
            startc(j, j)
        g_issue(0, 0)
        g_issue(1, 1)

        @pl.loop(0, QPW // nbuf)
        def _(kk):
            q0 = kk * nbuf
            for j in range(nbuf):
                ql = q0 + j
                compute(ql, j)

                @pl.when(ql + 2 < QPW)
                def _():
                    g_issue(ql + 2, (j + 2) % nbuf)

                @pl.when(ql + 4 < QPW)
                def _():
                    startc(ql + 4, j)

        for ql in range(QPW - nbuf, QPW):
            j = ql % nbuf
            pltpu.make_async_copy(
                obs[j], out_hbm.at[pl.ds((base + ql) * 2, 2)],
                oss[j]).wait()

    return k(table, comb)


def _layer_norm(x, g, b):
    m = jnp.mean(x, axis=-1, keepdims=True)
    v = jnp.mean((x - m) ** 2, axis=-1, keepdims=True)
    return (x - m) * jax.lax.rsqrt(v + 1e-5) * g + b


def _post_body(attn_ref, tgt_ref, wop_ref, bop_ref, g1_ref, b1_ref,
               wfc_ref, bfc_ref, wpr_ref, bpr_ref, g2_ref, b2_ref, out_ref):
    tgt2 = jnp.dot(attn_ref[0], wop_ref[...],
                   preferred_element_type=jnp.float32) + bop_ref[...]
    x = _layer_norm(tgt_ref[0] + tgt2, g1_ref[...], b1_ref[...])
    h = jnp.maximum(
        jnp.dot(x, wfc_ref[...], preferred_element_type=jnp.float32)
        + bfc_ref[...], 0.0)
    h = jnp.dot(h, wpr_ref[...], preferred_element_type=jnp.float32) + bpr_ref[...]
    out_ref[0] = _layer_norm(x + h, g2_ref[...], b2_ref[...])


def _post(attn, tgt_p, W_op, b_op, g1, b1, W_fc, b_fc, W_pr, b_pr, g2, b2):
    qspec = pl.BlockSpec((1, LQP, D_MODEL), lambda b: (b, 0, 0))
    return pl.pallas_call(
        _post_body,
        grid=(B,),
        in_specs=[
            qspec, qspec,
            pl.BlockSpec((D_MODEL, D_MODEL), lambda b: (0, 0)),
            pl.BlockSpec((1, D_MODEL), lambda b: (0, 0)),
            pl.BlockSpec((1, D_MODEL), lambda b: (0, 0)),
            pl.BlockSpec((1, D_MODEL), lambda b: (0, 0)),
            pl.BlockSpec((D_MODEL, 4 * D_MODEL), lambda b: (0, 0)),
            pl.BlockSpec((1, 4 * D_MODEL), lambda b: (0, 0)),
            pl.BlockSpec((4 * D_MODEL, D_MODEL), lambda b: (0, 0)),
            pl.BlockSpec((1, D_MODEL), lambda b: (0, 0)),
            pl.BlockSpec((1, D_MODEL), lambda b: (0, 0)),
            pl.BlockSpec((1, D_MODEL), lambda b: (0, 0)),
        ],
        out_specs=qspec,
        out_shape=jax.ShapeDtypeStruct((B, LQP, D_MODEL), jnp.float32),
    )(attn, tgt_p, W_op, b_op, g1, b1, W_fc, b_fc, W_pr, b_pr, g2, b2)


def kernel(tgt, query_pos, reference_points, src, src_spatial_shapes,
           level_start_index, src_padding_mask, W_so, b_so, W_aw, b_aw,
           W_vp, b_vp, W_op, b_op, ln1_g, ln1_b, W_fc, b_fc, W_pr, b_pr,
           ln2_g, ln2_b):
    maskf = (1.0 - src_padding_mask.astype(jnp.float32))[..., None]
    value = _value_proj(src, W_vp, b_vp.reshape(1, -1), maskf)
    table = value.reshape(B * LEN_PAD * N_HEAD, HEAD_DIM)

    pad_q = [(0, 0), (0, LQP - LQ), (0, 0)]
    tgt_p = jnp.pad(tgt, pad_q)
    qpos_p = jnp.pad(query_pos, pad_q)
    ref_p = jnp.pad(reference_points.reshape(B, LQ, 8), pad_q)
    W_so_p = W_so[:, jnp.asarray(_perm)]
    b_so_p = b_so[jnp.asarray(_perm)].reshape(1, -1)

    outs = _qproj(tgt_p, qpos_p, ref_p, W_so_p, b_so_p, W_aw,
                  b_aw.reshape(1, -1))
    idx = jnp.stack(outs[:4], axis=2).reshape(NQ, 4, 128)
    wbits = jax.lax.bitcast_convert_type(
        jnp.stack(outs[4:], axis=2), jnp.int32).reshape(NQ, 4, 128)
    comb = jnp.concatenate([idx, wbits], axis=1)

    attn = _sc_attend(table, comb).reshape(B, LQP, D_MODEL)
    out = _post(attn, tgt_p, W_op, b_op.reshape(1, -1),
                ln1_g.reshape(1, -1), ln1_b.reshape(1, -1),
                W_fc, b_fc.reshape(1, -1), W_pr, b_pr.reshape(1, -1),
                ln2_g.reshape(1, -1), ln2_b.reshape(1, -1))
    return out[:, :LQ, :]


# speedup vs baseline: 874.9584x; 1.3927x over previous
import functools

import jax
import jax.numpy as jnp
import numpy as np
from jax.experimental import pallas as pl
from jax.experimental.pallas import tpu as pltpu
from jax.experimental.pallas import tpu_sc as plsc

D_MODEL = 256
N_HEAD = 8
N_LEVELS = 4
N_POINTS = 4
HEAD_DIM = 32
B = 4
LQ = 900
LQP = 928
SPATIAL_SHAPES = [(100, 100), (50, 50), (25, 25), (13, 13)]
LEVEL_START = [0, 10000, 12500, 13125]
LEN_IN = 13294
LEN_PAD = 13312
NCOL = 128
N_CORNER = 4
N_IDX = B * LQP * N_CORNER * NCOL

_m_of_col = np.arange(NCOL) // 16
_l_of_col = (np.arange(NCOL) // 4) % 4
_W_of_col = np.array([SPATIAL_SHAPES[l][1] for l in _l_of_col], np.float32)
_H_of_col = np.array([SPATIAL_SHAPES[l][0] for l in _l_of_col], np.float32)
_LS_of_col = np.array([LEVEL_START[l] for l in _l_of_col], np.int32)

WVEC = _W_of_col[None, :]
HVEC = _H_of_col[None, :]
WVEC_I = _W_of_col.astype(np.int32)[None, :]
LSVEC_I = _LS_of_col[None, :]
MVEC_I = _m_of_col.astype(np.int32)[None, :]

_SelX = np.zeros((8, NCOL), np.float32)
_SelY = np.zeros((8, NCOL), np.float32)
for c in range(NCOL):
    l = _l_of_col[c]
    _SelX[2 * l + 0, c] = SPATIAL_SHAPES[l][1]
    _SelY[2 * l + 1, c] = SPATIAL_SHAPES[l][0]

_GM = np.kron(np.eye(8, dtype=np.float32), np.ones((16, 16), np.float32))

_perm = np.empty(256, np.int64)
for c in range(NCOL):
    m, l, p = c // 16, (c // 4) % 4, c % 4
    _perm[c] = ((m * 4 + l) * 4 + p) * 2
    _perm[128 + c] = ((m * 4 + l) * 4 + p) * 2 + 1


def _value_proj_body(src_ref, w_ref, b_ref, mask_ref, out_ref):
    v = jnp.dot(src_ref[0], w_ref[...], preferred_element_type=jnp.float32)
    v = (v + b_ref[...]) * mask_ref[0]
    lo = jnp.concatenate([v[:, m * 32:m * 32 + 16] for m in range(8)], axis=1)
    hi = jnp.concatenate([v[:, m * 32 + 16:m * 32 + 32] for m in range(8)],
                         axis=1)

    def _rne(x):
        xb = jax.lax.bitcast_convert_type(x, jnp.int32)
        return xb + 0x7FFF + ((xb >> 16) & 1)

    out_ref[...] = jax.lax.shift_right_logical(_rne(lo), 16) | (
        _rne(hi) & -65536)


def _value_proj(src, W_vp, b_vp, maskf):
    n_blk = LEN_PAD // 256
    return pl.pallas_call(
        _value_proj_body,
        grid=(B, n_blk),
        in_specs=[
            pl.BlockSpec((1, 256, D_MODEL), lambda b, i: (b, i, 0)),
            pl.BlockSpec((D_MODEL, D_MODEL), lambda b, i: (0, 0)),
            pl.BlockSpec((1, D_MODEL), lambda b, i: (0, 0)),
            pl.BlockSpec((1, 256, 1), lambda b, i: (b, i, 0)),
        ],
        out_specs=pl.BlockSpec((256, 128), lambda b, i: (b * n_blk + i, 0)),
        out_shape=jax.ShapeDtypeStruct((B * LEN_PAD, 128), jnp.int32),
    )(src, W_vp, b_vp, maskf)


def _qproj_body(tgt_ref, qpos_ref, ref_ref, wso_ref, bso_ref, waw_ref,
                baw_ref, gm_ref, sel_ref, cst_ref, *out_refs):
    b = pl.program_id(0)
    q = tgt_ref[0] + qpos_ref[0]
    so = jnp.dot(q, wso_ref[...], preferred_element_type=jnp.float32) + bso_ref[...]
    logits = jnp.dot(q, waw_ref[...], preferred_element_type=jnp.float32) + baw_ref[...]
    e = jnp.exp(logits - jnp.max(logits, axis=-1, keepdims=True))
    denom = jnp.dot(e, gm_ref[...], preferred_element_type=jnp.float32)
    aw = e / denom

    x = jnp.dot(ref_ref[0], sel_ref[:8],
                preferred_element_type=jnp.float32) + so[:, :NCOL] - 0.5
    y = jnp.dot(ref_ref[0], sel_ref[8:],
                preferred_element_type=jnp.float32) + so[:, NCOL:] - 0.5
    x0 = jnp.floor(x)
    y0 = jnp.floor(y)
    fx = x - x0
    fy = y - y0
    wvec = cst_ref[0:1]
    hvec = cst_ref[1:2]
    wvec_i = wvec.astype(jnp.int32)
    lsm_i = (cst_ref[2:3] * N_HEAD + cst_ref[3:4]).astype(jnp.int32)
    base = b * (LEN_PAD * N_HEAD)
    out_ref = out_refs[0]
    for c, (dx, dy) in enumerate(((0, 0), (1, 0), (0, 1), (1, 1))):
        ix = x0 + dx
        iy = y0 + dy
        vx = ((ix >= 0.0) & (ix <= wvec - 1.0)).astype(jnp.float32)
        vy = ((iy >= 0.0) & (iy <= hvec - 1.0)).astype(jnp.float32)
        ixc = jnp.clip(ix, 0.0, wvec - 1.0).astype(jnp.int32)
        iyc = jnp.clip(iy, 0.0, hvec - 1.0).astype(jnp.int32)
        wx = fx if dx else 1.0 - fx
        wy = fy if dy else 1.0 - fy
        sp = iyc * wvec_i + ixc
        out_ref[0, :, c, :] = base + sp * N_HEAD + lsm_i
        out_ref[0, :, 4 + c, :] = jax.lax.bitcast_convert_type(
            wx * wy * vx * vy * aw, jnp.int32)


def _qproj(tgt_p, qpos_p, ref_p, W_so_p, b_so_p, W_aw, b_aw):
    qspec = pl.BlockSpec((1, LQP, D_MODEL), lambda b: (b, 0, 0))
    ospec = pl.BlockSpec((1, LQP, 8, NCOL), lambda b: (b, 0, 0, 0))
    oshape = jax.ShapeDtypeStruct((B, LQP, 8, NCOL), jnp.int32)
    gm = jnp.asarray(_GM)
    sel = jnp.asarray(np.concatenate([_SelX, _SelY], axis=0))
    cst = jnp.asarray(np.stack([_W_of_col, _H_of_col,
                                _LS_of_col.astype(np.float32),
                                _m_of_col.astype(np.float32)], axis=0))
    return pl.pallas_call(
        _qproj_body,
        grid=(B,),
        in_specs=[
            qspec, qspec,
            pl.BlockSpec((1, LQP, 8), lambda b: (b, 0, 0)),
            pl.BlockSpec((D_MODEL, 256), lambda b: (0, 0)),
            pl.BlockSpec((1, 256), lambda b: (0, 0)),
            pl.BlockSpec((D_MODEL, NCOL), lambda b: (0, 0)),
            pl.BlockSpec((1, NCOL), lambda b: (0, 0)),
            pl.BlockSpec((NCOL, NCOL), lambda b: (0, 0)),
            pl.BlockSpec((16, NCOL), lambda b: (0, 0)),
            pl.BlockSpec((4, NCOL), lambda b: (0, 0)),
        ],
        out_specs=ospec,
        out_shape=oshape,
    )(tgt_p, qpos_p, ref_p, W_so_p, b_so_p, W_aw, b_aw, gm, sel, cst)


NW = 32
NQ = B * LQP
QPW = NQ // NW
NT = N_CORNER * NCOL


def _sc_attend(table, comb):
    mesh = plsc.VectorSubcoreMesh(core_axis_name="c", subcore_axis_name="s")

    nbuf = 4

    @functools.partial(
        pl.kernel,
        out_type=jax.ShapeDtypeStruct((NQ * 2, 128), jnp.float32),
        mesh=mesh,
        compiler_params=pltpu.CompilerParams(use_tc_tiling_on_sc=False,
                                             needs_layout_passes=False),
        scratch_types=(
            [pltpu.VMEM((8, 128), jnp.int32)] * nbuf
            + [pltpu.VMEM((NT, 16), jnp.int32)] * nbuf
            + [pltpu.VMEM((2, 128), jnp.float32)] * nbuf
            + [pltpu.SemaphoreType.DMA] * (3 * nbuf)
        ),
    )
    def k(table_hbm, comb_hbm, out_hbm, *scr):
        cbs, gbs, obs = scr[0:4], scr[4:8], scr[8:12]
        css, gss, oss = scr[12:16], scr[16:20], scr[20:24]
        iota = jax.lax.iota(jnp.int32, 16)
        wid = jax.lax.axis_index("s") * 2 + jax.lax.axis_index("c")
        base = wid * QPW

        def startc(ql, j):
            pltpu.make_async_copy(comb_hbm.at[base + ql], cbs[j],
                                  css[j]).start()

        def g_issue(ql, j):
            pltpu.make_async_copy(comb_hbm.at[base + ql], cbs[j],
                                  css[j]).wait()
            for c in range(N_CORNER):
                pltpu.make_async_copy(table_hbm.at[cbs[j].at[c]],
                                      gbs[j].at[pl.ds(c * 128, 128)],
                                      gss[j]).start()

        def compute(ql, j):
            for c in range(N_CORNER):
                pltpu.make_async_copy(table_hbm.at[cbs[j].at[c]],
                                      gbs[j].at[pl.ds(c * 128, 128)],
                                      gss[j]).wait()

            @pl.when(ql >= nbuf)
            def _():
                pltpu.make_async_copy(
                    obs[j], out_hbm.at[pl.ds((base + ql) * 2, 2)],
                    oss[j]).wait()

            @pl.loop(0, N_HEAD)
            def _(m):
                acc_e = jnp.zeros((16,), jnp.float32)
                acc_o = jnp.zeros((16,), jnp.float32)
                for c in range(N_CORNER):
                    for t in range(16):
                        off2 = m * 16 + t
                        wi = plsc.load_gather(
                            cbs[j], [jnp.full((16,), 4 + c, jnp.int32),
                                     jnp.full((16,), off2, jnp.int32)])
                        wf = plsc.bitcast(wi, jnp.float32)
                        rows = jnp.full((16,), c * 128 + off2, jnp.int32)
                        gi = plsc.load_gather(gbs[j], [rows, iota])
                        ge = plsc.bitcast(jax.lax.shift_left(gi, 16),
                                          jnp.float32)
                        go = plsc.bitcast(gi, jnp.float32)
                        acc_e = acc_e + wf * ge
                        acc_o = acc_o + wf * go
                orow0 = jnp.full((16,), m // 4, jnp.int32)
                ocol = (m % 4) * 32 + iota
                plsc.store_scatter(obs[j], [orow0, ocol], acc_e)
                plsc.store_scatter(obs[j], [orow0, ocol + 16], acc_o)

            pltpu.make_async_copy(
                obs[j], out_hbm.at[pl.ds((base + ql) * 2, 2)],
                oss[j]).start()

        for j in range(nbuf):
            startc(j, j)
        g_issue(0, 0)
        g_issue(1, 1)

        @pl.loop(0, QPW // nbuf)
        def _(kk):
            q0 = kk * nbuf
            for j in range(nbuf):
                ql = q0 + j
                compute(ql, j)

                @pl.when(ql + 2 < QPW)
                def _():
                    g_issue(ql + 2, (j + 2) % nbuf)

                @pl.when(ql + 4 < QPW)
                def _():
                    startc(ql + 4, j)

        for ql in range(QPW - nbuf, QPW):
            j = ql % nbuf
            pltpu.make_async_copy(
                obs[j], out_hbm.at[pl.ds((base + ql) * 2, 2)],
                oss[j]).wait()

    return k(table, comb)


def _layer_norm(x, g, b):
    m = jnp.mean(x, axis=-1, keepdims=True)
    v = jnp.mean((x - m) ** 2, axis=-1, keepdims=True)
    return (x - m) * jax.lax.rsqrt(v + 1e-5) * g + b


def _post_body(attn_ref, tgt_ref, wop_ref, bop_ref, g1_ref, b1_ref,
               wfc_ref, bfc_ref, wpr_ref, bpr_ref, g2_ref, b2_ref, out_ref):
    tgt2 = jnp.dot(attn_ref[0], wop_ref[...],
                   preferred_element_type=jnp.float32) + bop_ref[...]
    x = _layer_norm(tgt_ref[0] + tgt2, g1_ref[...], b1_ref[...])
    h = jnp.maximum(
        jnp.dot(x, wfc_ref[...], preferred_element_type=jnp.float32)
        + bfc_ref[...], 0.0)
    h = jnp.dot(h, wpr_ref[...], preferred_element_type=jnp.float32) + bpr_ref[...]
    out_ref[0] = _layer_norm(x + h, g2_ref[...], b2_ref[...])


def _post(attn, tgt_p, W_op, b_op, g1, b1, W_fc, b_fc, W_pr, b_pr, g2, b2):
    qspec = pl.BlockSpec((1, LQP, D_MODEL), lambda b: (b, 0, 0))
    return pl.pallas_call(
        _post_body,
        grid=(B,),
        in_specs=[
            qspec, qspec,
            pl.BlockSpec((D_MODEL, D_MODEL), lambda b: (0, 0)),
            pl.BlockSpec((1, D_MODEL), lambda b: (0, 0)),
            pl.BlockSpec((1, D_MODEL), lambda b: (0, 0)),
            pl.BlockSpec((1, D_MODEL), lambda b: (0, 0)),
            pl.BlockSpec((D_MODEL, 4 * D_MODEL), lambda b: (0, 0)),
            pl.BlockSpec((1, 4 * D_MODEL), lambda b: (0, 0)),
            pl.BlockSpec((4 * D_MODEL, D_MODEL), lambda b: (0, 0)),
            pl.BlockSpec((1, D_MODEL), lambda b: (0, 0)),
            pl.BlockSpec((1, D_MODEL), lambda b: (0, 0)),
            pl.BlockSpec((1, D_MODEL), lambda b: (0, 0)),
        ],
        out_specs=qspec,
        out_shape=jax.ShapeDtypeStruct((B, LQP, D_MODEL), jnp.float32),
    )(attn, tgt_p, W_op, b_op, g1, b1, W_fc, b_fc, W_pr, b_pr, g2, b2)


def kernel(tgt, query_pos, reference_points, src, src_spatial_shapes,
           level_start_index, src_padding_mask, W_so, b_so, W_aw, b_aw,
           W_vp, b_vp, W_op, b_op, ln1_g, ln1_b, W_fc, b_fc, W_pr, b_pr,
           ln2_g, ln2_b):
    maskf = (1.0 - src_padding_mask.astype(jnp.float32))[..., None]
    value = _value_proj(src, W_vp, b_vp.reshape(1, -1), maskf)
    table = value.reshape(B * LEN_PAD * N_HEAD, 16)

    pad_q = [(0, 0), (0, LQP - LQ), (0, 0)]
    tgt_p = jnp.pad(tgt, pad_q)
    qpos_p = jnp.pad(query_pos, pad_q)
    ref_p = jnp.pad(reference_points.reshape(B, LQ, 8), pad_q)
    W_so_p = W_so[:, jnp.asarray(_perm)]
    b_so_p = b_so[jnp.asarray(_perm)].reshape(1, -1)

    qout = _qproj(tgt_p, qpos_p, ref_p, W_so_p, b_so_p, W_aw,
                  b_aw.reshape(1, -1))
    comb = qout.reshape(NQ, 8, NCOL)

    attn = _sc_attend(table, comb).reshape(B, LQP, D_MODEL)
    out = _post(attn, tgt_p, W_op, b_op.reshape(1, -1),
                ln1_g.reshape(1, -1), ln1_b.reshape(1, -1),
                W_fc, b_fc.reshape(1, -1), W_pr, b_pr.reshape(1, -1),
                ln2_g.reshape(1, -1), ln2_b.reshape(1, -1))
    return out[:, :LQ, :]


# speedup vs baseline: 915.1436x; 1.0459x over previous
import functools

import jax
import jax.numpy as jnp
import numpy as np
from jax.experimental import pallas as pl
from jax.experimental.pallas import tpu as pltpu
from jax.experimental.pallas import tpu_sc as plsc

D_MODEL = 256
N_HEAD = 8
N_LEVELS = 4
N_POINTS = 4
HEAD_DIM = 32
B = 4
LQ = 900
LQP = 928
SPATIAL_SHAPES = [(100, 100), (50, 50), (25, 25), (13, 13)]
LEVEL_START = [0, 10000, 12500, 13125]
LEN_IN = 13294
LEN_PAD = 13312
NCOL = 128
N_CORNER = 4
N_IDX = B * LQP * N_CORNER * NCOL

_m_of_col = np.arange(NCOL) // 16
_l_of_col = (np.arange(NCOL) // 4) % 4
_W_of_col = np.array([SPATIAL_SHAPES[l][1] for l in _l_of_col], np.float32)
_H_of_col = np.array([SPATIAL_SHAPES[l][0] for l in _l_of_col], np.float32)
_LS_of_col = np.array([LEVEL_START[l] for l in _l_of_col], np.int32)

WVEC = _W_of_col[None, :]
HVEC = _H_of_col[None, :]
WVEC_I = _W_of_col.astype(np.int32)[None, :]
LSVEC_I = _LS_of_col[None, :]
MVEC_I = _m_of_col.astype(np.int32)[None, :]

_SelX = np.zeros((8, NCOL), np.float32)
_SelY = np.zeros((8, NCOL), np.float32)
for c in range(NCOL):
    l = _l_of_col[c]
    _SelX[2 * l + 0, c] = SPATIAL_SHAPES[l][1]
    _SelY[2 * l + 1, c] = SPATIAL_SHAPES[l][0]

_GM = np.kron(np.eye(8, dtype=np.float32), np.ones((16, 16), np.float32))

_perm = np.empty(256, np.int64)
for c in range(NCOL):
    m, l, p = c // 16, (c // 4) % 4, c % 4
    _perm[c] = ((m * 4 + l) * 4 + p) * 2
    _perm[128 + c] = ((m * 4 + l) * 4 + p) * 2 + 1

_permv = np.empty(256, np.int64)
for m in range(8):
    for k in range(16):
        _permv[m * 16 + k] = m * 32 + k
        _permv[128 + m * 16 + k] = m * 32 + 16 + k


def _value_proj_body(src_ref, w_ref, b_ref, mask_ref, out_ref):
    v = jnp.dot(src_ref[0], w_ref[...], preferred_element_type=jnp.float32)
    v = (v + b_ref[...]) * mask_ref[0]

    def _rne(x):
        xb = jax.lax.bitcast_convert_type(x, jnp.int32)
        return xb + 0x7FFF + ((xb >> 16) & 1)

    out_ref[...] = jax.lax.shift_right_logical(_rne(v[:, :128]), 16) | (
        _rne(v[:, 128:]) & -65536)


def _value_proj(src, W_vp, b_vp, maskf):
    n_blk = LEN_PAD // 256
    return pl.pallas_call(
        _value_proj_body,
        grid=(B, n_blk),
        in_specs=[
            pl.BlockSpec((1, 256, D_MODEL), lambda b, i: (b, i, 0)),
            pl.BlockSpec((D_MODEL, D_MODEL), lambda b, i: (0, 0)),
            pl.BlockSpec((1, D_MODEL), lambda b, i: (0, 0)),
            pl.BlockSpec((1, 256, 1), lambda b, i: (b, i, 0)),
        ],
        out_specs=pl.BlockSpec((256, 128), lambda b, i: (b * n_blk + i, 0)),
        out_shape=jax.ShapeDtypeStruct((B * LEN_PAD, 128), jnp.int32),
    )(src.astype(jnp.bfloat16), W_vp, b_vp, maskf)


def _qproj_body(tgt_ref, qpos_ref, ref_ref, wso_ref, bso_ref, waw_ref,
                baw_ref, gm_ref, sel_ref, cst_ref, *out_refs):
    b = pl.program_id(0)
    q = tgt_ref[0] + qpos_ref[0]
    so = jnp.dot(q, wso_ref[...], preferred_element_type=jnp.float32) + bso_ref[...]
    logits = jnp.dot(q, waw_ref[...], preferred_element_type=jnp.float32) + baw_ref[...]
    e = jnp.exp(logits - jnp.max(logits, axis=-1, keepdims=True))
    denom = jnp.dot(e, gm_ref[...], preferred_element_type=jnp.float32)
    aw = e / denom

    x = jnp.dot(ref_ref[0], sel_ref[:8],
                preferred_element_type=jnp.float32) + so[:, :NCOL] - 0.5
    y = jnp.dot(ref_ref[0], sel_ref[8:],
                preferred_element_type=jnp.float32) + so[:, NCOL:] - 0.5
    x0 = jnp.floor(x)
    y0 = jnp.floor(y)
    fx = x - x0
    fy = y - y0
    wvec = cst_ref[0:1]
    hvec = cst_ref[1:2]
    wvec_i = wvec.astype(jnp.int32)
    lsm_i = (cst_ref[2:3] * N_HEAD + cst_ref[3:4]).astype(jnp.int32)
    base = b * (LEN_PAD * N_HEAD)
    out_ref = out_refs[0]
    for c, (dx, dy) in enumerate(((0, 0), (1, 0), (0, 1), (1, 1))):
        ix = x0 + dx
        iy = y0 + dy
        vx = ((ix >= 0.0) & (ix <= wvec - 1.0)).astype(jnp.float32)
        vy = ((iy >= 0.0) & (iy <= hvec - 1.0)).astype(jnp.float32)
        ixc = jnp.clip(ix, 0.0, wvec - 1.0).astype(jnp.int32)
        iyc = jnp.clip(iy, 0.0, hvec - 1.0).astype(jnp.int32)
        wx = fx if dx else 1.0 - fx
        wy = fy if dy else 1.0 - fy
        sp = iyc * wvec_i + ixc
        out_ref[0, :, c, :] = base + sp * N_HEAD + lsm_i
        out_ref[0, :, 4 + c, :] = jax.lax.bitcast_convert_type(
            wx * wy * vx * vy * aw, jnp.int32)


def _qproj(tgt_p, qpos_p, ref_p, W_so_p, b_so_p, W_aw, b_aw):
    qspec = pl.BlockSpec((1, LQP, D_MODEL), lambda b: (b, 0, 0))
    ospec = pl.BlockSpec((1, LQP, 8, NCOL), lambda b: (b, 0, 0, 0))
    oshape = jax.ShapeDtypeStruct((B, LQP, 8, NCOL), jnp.int32)
    gm = jnp.asarray(_GM)
    sel = jnp.asarray(np.concatenate([_SelX, _SelY], axis=0))
    cst = jnp.asarray(np.stack([_W_of_col, _H_of_col,
                                _LS_of_col.astype(np.float32),
                                _m_of_col.astype(np.float32)], axis=0))
    return pl.pallas_call(
        _qproj_body,
        grid=(B,),
        in_specs=[
            qspec, qspec,
            pl.BlockSpec((1, LQP, 8), lambda b: (b, 0, 0)),
            pl.BlockSpec((D_MODEL, 256), lambda b: (0, 0)),
            pl.BlockSpec((1, 256), lambda b: (0, 0)),
            pl.BlockSpec((D_MODEL, NCOL), lambda b: (0, 0)),
            pl.BlockSpec((1, NCOL), lambda b: (0, 0)),
            pl.BlockSpec((NCOL, NCOL), lambda b: (0, 0)),
            pl.BlockSpec((16, NCOL), lambda b: (0, 0)),
            pl.BlockSpec((4, NCOL), lambda b: (0, 0)),
        ],
        out_specs=ospec,
        out_shape=oshape,
    )(tgt_p, qpos_p, ref_p, W_so_p, b_so_p, W_aw, b_aw, gm, sel, cst)


NW = 32
NQ = B * LQP
QPW = NQ // NW
NT = N_CORNER * NCOL


def _sc_attend(table, comb):
    mesh = plsc.VectorSubcoreMesh(core_axis_name="c", subcore_axis_name="s")

    nbuf = 4

    @functools.partial(
        pl.kernel,
        out_type=jax.ShapeDtypeStruct((NQ * 2, 128), jnp.float32),
        mesh=mesh,
        compiler_params=pltpu.CompilerParams(use_tc_tiling_on_sc=False,
                                             needs_layout_passes=False),
        scratch_types=(
            [pltpu.VMEM((8, 128), jnp.int32)] * nbuf
            + [pltpu.VMEM((NT, 16), jnp.int32)] * nbuf
            + [pltpu.VMEM((2, 128), jnp.float32)] * nbuf
            + [pltpu.SemaphoreType.DMA] * (3 * nbuf)
        ),
    )
    def k(table_hbm, comb_hbm, out_hbm, *scr):
        cbs, gbs, obs = scr[0:4], scr[4:8], scr[8:12]
        css, gss, oss = scr[12:16], scr[16:20], scr[20:24]
        iota = jax.lax.iota(jnp.int32, 16)
        wid = jax.lax.axis_index("s") * 2 + jax.lax.axis_index("c")
        base = wid * QPW

        def startc(ql, j):
            pltpu.make_async_copy(comb_hbm.at[base + ql], cbs[j],
                                  css[j]).start()

        def g_issue(ql, j):
            pltpu.make_async_copy(comb_hbm.at[base + ql], cbs[j],
                                  css[j]).wait()
            for c in range(N_CORNER):
                pltpu.make_async_copy(table_hbm.at[cbs[j].at[c]],
                                      gbs[j].at[pl.ds(c * 128, 128)],
                                      gss[j]).start()

        def compute(ql, j):
            for c in range(N_CORNER):
                pltpu.make_async_copy(table_hbm.at[cbs[j].at[c]],
                                      gbs[j].at[pl.ds(c * 128, 128)],
                                      gss[j]).wait()

            @pl.when(ql >= nbuf)
            def _():
                pltpu.make_async_copy(
                    obs[j], out_hbm.at[pl.ds((base + ql) * 2, 2)],
                    oss[j]).wait()

            @pl.loop(0, N_HEAD)
            def _(m):
                acc_e = jnp.zeros((16,), jnp.float32)
                acc_o = jnp.zeros((16,), jnp.float32)
                for c in range(N_CORNER):
                    for t in range(16):
                        off2 = m * 16 + t
                        wi = plsc.load_gather(
                            cbs[j], [jnp.full((16,), 4 + c, jnp.int32),
                                     jnp.full((16,), off2, jnp.int32)])
                        wf = plsc.bitcast(wi, jnp.float32)
                        rows = jnp.full((16,), c * 128 + off2, jnp.int32)
                        gi = plsc.load_gather(gbs[j], [rows, iota])
                        ge = plsc.bitcast(jax.lax.shift_left(gi, 16),
                                          jnp.float32)
                        go = plsc.bitcast(gi, jnp.float32)
                        acc_e = acc_e + wf * ge
                        acc_o = acc_o + wf * go
                orow0 = jnp.full((16,), m // 4, jnp.int32)
                ocol = (m % 4) * 32 + iota
                plsc.store_scatter(obs[j], [orow0, ocol], acc_e)
                plsc.store_scatter(obs[j], [orow0, ocol + 16], acc_o)

            pltpu.make_async_copy(
                obs[j], out_hbm.at[pl.ds((base + ql) * 2, 2)],
                oss[j]).start()

        for j in range(nbuf):
            startc(j, j)
        g_issue(0, 0)
        g_issue(1, 1)

        @pl.loop(0, QPW // nbuf)
        def _(kk):
            q0 = kk * nbuf
            for j in range(nbuf):
                ql = q0 + j
                compute(ql, j)

                @pl.when(ql + 2 < QPW)
                def _():
                    g_issue(ql + 2, (j + 2) % nbuf)

                @pl.when(ql + 4 < QPW)
                def _():
                    startc(ql + 4, j)

        for ql in range(QPW - nbuf, QPW):
            j = ql % nbuf
            pltpu.make_async_copy(
                obs[j], out_hbm.at[pl.ds((base + ql) * 2, 2)],
                oss[j]).wait()

    return k(table, comb)


def _layer_norm(x, g, b):
    m = jnp.mean(x, axis=-1, keepdims=True)
    v = jnp.mean((x - m) ** 2, axis=-1, keepdims=True)
    return (x - m) * jax.lax.rsqrt(v + 1e-5) * g + b


def _post_body(attn_ref, tgt_ref, wop_ref, bop_ref, g1_ref, b1_ref,
               wfc_ref, bfc_ref, wpr_ref, bpr_ref, g2_ref, b2_ref, out_ref):
    tgt2 = jnp.dot(attn_ref[0], wop_ref[...],
                   preferred_element_type=jnp.float32) + bop_ref[...]
    x = _layer_norm(tgt_ref[0] + tgt2, g1_ref[...], b1_ref[...])
    h = jnp.maximum(
        jnp.dot(x, wfc_ref[...], preferred_element_type=jnp.float32)
        + bfc_ref[...], 0.0)
    h = jnp.dot(h, wpr_ref[...], preferred_element_type=jnp.float32) + bpr_ref[...]
    out_ref[0] = _layer_norm(x + h, g2_ref[...], b2_ref[...])


def _post(attn, tgt_p, W_op, b_op, g1, b1, W_fc, b_fc, W_pr, b_pr, g2, b2):
    qspec = pl.BlockSpec((1, LQP, D_MODEL), lambda b: (b, 0, 0))
    return pl.pallas_call(
        _post_body,
        grid=(B,),
        in_specs=[
            qspec, qspec,
            pl.BlockSpec((D_MODEL, D_MODEL), lambda b: (0, 0)),
            pl.BlockSpec((1, D_MODEL), lambda b: (0, 0)),
            pl.BlockSpec((1, D_MODEL), lambda b: (0, 0)),
            pl.BlockSpec((1, D_MODEL), lambda b: (0, 0)),
            pl.BlockSpec((D_MODEL, 4 * D_MODEL), lambda b: (0, 0)),
            pl.BlockSpec((1, 4 * D_MODEL), lambda b: (0, 0)),
            pl.BlockSpec((4 * D_MODEL, D_MODEL), lambda b: (0, 0)),
            pl.BlockSpec((1, D_MODEL), lambda b: (0, 0)),
            pl.BlockSpec((1, D_MODEL), lambda b: (0, 0)),
            pl.BlockSpec((1, D_MODEL), lambda b: (0, 0)),
        ],
        out_specs=qspec,
        out_shape=jax.ShapeDtypeStruct((B, LQP, D_MODEL), jnp.float32),
    )(attn, tgt_p, W_op, b_op, g1, b1, W_fc, b_fc, W_pr, b_pr, g2, b2)


def kernel(tgt, query_pos, reference_points, src, src_spatial_shapes,
           level_start_index, src_padding_mask, W_so, b_so, W_aw, b_aw,
           W_vp, b_vp, W_op, b_op, ln1_g, ln1_b, W_fc, b_fc, W_pr, b_pr,
           ln2_g, ln2_b):
    maskf = (1.0 - src_padding_mask.astype(jnp.float32))[..., None]
    permv = jnp.asarray(_permv)
    value = _value_proj(src, W_vp[:, permv].astype(jnp.bfloat16),
                        b_vp[permv].reshape(1, -1), maskf)
    table = value.reshape(B * LEN_PAD * N_HEAD, 16)

    pad_q = [(0, 0), (0, LQP - LQ), (0, 0)]
    tgt_p = jnp.pad(tgt, pad_q)
    qpos_p = jnp.pad(query_pos, pad_q)
    ref_p = jnp.pad(reference_points.reshape(B, LQ, 8), pad_q)
    W_so_p = W_so[:, jnp.asarray(_perm)]
    b_so_p = b_so[jnp.asarray(_perm)].reshape(1, -1)

    qout = _qproj(tgt_p, qpos_p, ref_p, W_so_p, b_so_p, W_aw,
                  b_aw.reshape(1, -1))
    comb = qout.reshape(NQ, 8, NCOL)

    attn = _sc_attend(table, comb).reshape(B, LQP, D_MODEL)
    out = _post(attn, tgt_p, W_op, b_op.reshape(1, -1),
                ln1_g.reshape(1, -1), ln1_b.reshape(1, -1),
                W_fc, b_fc.reshape(1, -1), W_pr, b_pr.reshape(1, -1),
                ln2_g.reshape(1, -1), ln2_b.reshape(1, -1))
    return out[:, :LQ, :]


# speedup vs baseline: 1122.0616x; 1.2261x over previous
import functools

import jax
import jax.numpy as jnp
import numpy as np
from jax.experimental import pallas as pl
from jax.experimental.pallas import tpu as pltpu
from jax.experimental.pallas import tpu_sc as plsc

D_MODEL = 256
N_HEAD = 8
N_LEVELS = 4
N_POINTS = 4
HEAD_DIM = 32
B = 4
LQ = 900
LQP = 928
SPATIAL_SHAPES = [(100, 100), (50, 50), (25, 25), (13, 13)]
LEVEL_START = [0, 10000, 12500, 13125]
LEN_IN = 13294
LEN_PAD = 13312
NCOL = 128
N_CORNER = 4
N_IDX = B * LQP * N_CORNER * NCOL

_m_of_col = np.arange(NCOL) // 16
_l_of_col = (np.arange(NCOL) // 4) % 4
_W_of_col = np.array([SPATIAL_SHAPES[l][1] for l in _l_of_col], np.float32)
_H_of_col = np.array([SPATIAL_SHAPES[l][0] for l in _l_of_col], np.float32)
_LS_of_col = np.array([LEVEL_START[l] for l in _l_of_col], np.int32)

WVEC = _W_of_col[None, :]
HVEC = _H_of_col[None, :]
WVEC_I = _W_of_col.astype(np.int32)[None, :]
LSVEC_I = _LS_of_col[None, :]
MVEC_I = _m_of_col.astype(np.int32)[None, :]

_SelX = np.zeros((8, NCOL), np.float32)
_SelY = np.zeros((8, NCOL), np.float32)
for c in range(NCOL):
    l = _l_of_col[c]
    _SelX[2 * l + 0, c] = SPATIAL_SHAPES[l][1]
    _SelY[2 * l + 1, c] = SPATIAL_SHAPES[l][0]

_GM = np.kron(np.eye(8, dtype=np.float32), np.ones((16, 16), np.float32))

_perm = np.empty(256, np.int64)
for c in range(NCOL):
    m, l, p = c // 16, (c // 4) % 4, c % 4
    _perm[c] = ((m * 4 + l) * 4 + p) * 2
    _perm[128 + c] = ((m * 4 + l) * 4 + p) * 2 + 1

_permv = np.empty(256, np.int64)
for m in range(8):
    for k in range(16):
        _permv[m * 16 + k] = m * 32 + k
        _permv[128 + m * 16 + k] = m * 32 + 16 + k


_VROWS = 1024


def _value_proj_body(src_ref, w_ref, b_ref, mask_ref, out_ref):
    v = jnp.dot(src_ref[0], w_ref[...], preferred_element_type=jnp.float32)
    v = (v + b_ref[...]) * mask_ref[0]

    def _rne(x):
        xb = jax.lax.bitcast_convert_type(x, jnp.int32)
        return xb + 0x7FFF + ((xb >> 16) & 1)

    out_ref[...] = jax.lax.shift_right_logical(_rne(v[:, :128]), 16) | (
        _rne(v[:, 128:]) & -65536)


def _value_proj(src, W_vp, b_vp, maskf):
    n_blk = LEN_PAD // _VROWS
    return pl.pallas_call(
        _value_proj_body,
        grid=(B, n_blk),
        in_specs=[
            pl.BlockSpec((1, _VROWS, D_MODEL), lambda b, i: (b, i, 0)),
            pl.BlockSpec((D_MODEL, D_MODEL), lambda b, i: (0, 0)),
            pl.BlockSpec((1, D_MODEL), lambda b, i: (0, 0)),
            pl.BlockSpec((1, _VROWS, 1), lambda b, i: (b, i, 0)),
        ],
        out_specs=pl.BlockSpec((_VROWS, 128), lambda b, i: (b * n_blk + i, 0)),
        out_shape=jax.ShapeDtypeStruct((B * LEN_PAD, 128), jnp.int32),
    )(src.astype(jnp.bfloat16), W_vp, b_vp, maskf)


def _qproj_body(tgt_ref, qpos_ref, ref_ref, wso_ref, bso_ref, waw_ref,
                baw_ref, gm_ref, sel_ref, cst_ref, *out_refs):
    b = pl.program_id(0)
    q = tgt_ref[0] + qpos_ref[0]
    so = jnp.dot(q, wso_ref[...], preferred_element_type=jnp.float32) + bso_ref[...]
    logits = jnp.dot(q, waw_ref[...], preferred_element_type=jnp.float32) + baw_ref[...]
    e = jnp.exp(logits - jnp.max(logits, axis=-1, keepdims=True))
    denom = jnp.dot(e, gm_ref[...], preferred_element_type=jnp.float32)
    aw = e / denom

    x = jnp.dot(ref_ref[0], sel_ref[:8],
                preferred_element_type=jnp.float32) + so[:, :NCOL] - 0.5
    y = jnp.dot(ref_ref[0], sel_ref[8:],
                preferred_element_type=jnp.float32) + so[:, NCOL:] - 0.5
    x0 = jnp.floor(x)
    y0 = jnp.floor(y)
    fx = x - x0
    fy = y - y0
    wvec = cst_ref[0:1]
    hvec = cst_ref[1:2]
    wvec_i = wvec.astype(jnp.int32)
    lsm_i = (cst_ref[2:3] * N_HEAD + cst_ref[3:4]).astype(jnp.int32)
    base = b * (LEN_PAD * N_HEAD)
    out_ref = out_refs[0]
    for c, (dx, dy) in enumerate(((0, 0), (1, 0), (0, 1), (1, 1))):
        ix = x0 + dx
        iy = y0 + dy
        vx = ((ix >= 0.0) & (ix <= wvec - 1.0)).astype(jnp.float32)
        vy = ((iy >= 0.0) & (iy <= hvec - 1.0)).astype(jnp.float32)
        ixc = jnp.clip(ix, 0.0, wvec - 1.0).astype(jnp.int32)
        iyc = jnp.clip(iy, 0.0, hvec - 1.0).astype(jnp.int32)
        wx = fx if dx else 1.0 - fx
        wy = fy if dy else 1.0 - fy
        sp = iyc * wvec_i + ixc
        out_ref[0, :, c, :] = base + sp * N_HEAD + lsm_i
        out_ref[0, :, 4 + c, :] = jax.lax.bitcast_convert_type(
            wx * wy * vx * vy * aw, jnp.int32)


def _qproj(tgt_p, qpos_p, ref_p, W_so_p, b_so_p, W_aw, b_aw):
    qspec = pl.BlockSpec((1, LQP, D_MODEL), lambda b: (b, 0, 0))
    ospec = pl.BlockSpec((1, LQP, 8, NCOL), lambda b: (b, 0, 0, 0))
    oshape = jax.ShapeDtypeStruct((B, LQP, 8, NCOL), jnp.int32)
    gm = jnp.asarray(_GM)
    sel = jnp.asarray(np.concatenate([_SelX, _SelY], axis=0))
    cst = jnp.asarray(np.stack([_W_of_col, _H_of_col,
                                _LS_of_col.astype(np.float32),
                                _m_of_col.astype(np.float32)], axis=0))
    return pl.pallas_call(
        _qproj_body,
        grid=(B,),
        in_specs=[
            qspec, qspec,
            pl.BlockSpec((1, LQP, 8), lambda b: (b, 0, 0)),
            pl.BlockSpec((D_MODEL, 256), lambda b: (0, 0)),
            pl.BlockSpec((1, 256), lambda b: (0, 0)),
            pl.BlockSpec((D_MODEL, NCOL), lambda b: (0, 0)),
            pl.BlockSpec((1, NCOL), lambda b: (0, 0)),
            pl.BlockSpec((NCOL, NCOL), lambda b: (0, 0)),
            pl.BlockSpec((16, NCOL), lambda b: (0, 0)),
            pl.BlockSpec((4, NCOL), lambda b: (0, 0)),
        ],
        out_specs=ospec,
        out_shape=oshape,
    )(tgt_p, qpos_p, ref_p, W_so_p, b_so_p, W_aw, b_aw, gm, sel, cst)


NW = 32
NQ = B * LQP
QPW = NQ // NW
NT = N_CORNER * NCOL


def _sc_attend(table, comb):
    mesh = plsc.VectorSubcoreMesh(core_axis_name="c", subcore_axis_name="s")

    nbuf = 4

    @functools.partial(
        pl.kernel,
        out_type=jax.ShapeDtypeStruct((NQ * 2, 128), jnp.float32),
        mesh=mesh,
        compiler_params=pltpu.CompilerParams(use_tc_tiling_on_sc=False,
                                             needs_layout_passes=False),
        scratch_types=(
            [pltpu.VMEM((8, 128), jnp.int32)] * nbuf
            + [pltpu.VMEM((NT, 16), jnp.int32)] * nbuf
            + [pltpu.VMEM((2, 128), jnp.float32)] * nbuf
            + [pltpu.SemaphoreType.DMA] * (3 * nbuf)
        ),
    )
    def k(table_hbm, comb_hbm, out_hbm, *scr):
        cbs, gbs, obs = scr[0:4], scr[4:8], scr[8:12]
        css, gss, oss = scr[12:16], scr[16:20], scr[20:24]
        iota = jax.lax.iota(jnp.int32, 16)
        wid = jax.lax.axis_index("s") * 2 + jax.lax.axis_index("c")
        base = wid * QPW

        def startc(ql, j):
            pltpu.make_async_copy(comb_hbm.at[base + ql], cbs[j],
                                  css[j]).start()

        def g_issue(ql, j):
            pltpu.make_async_copy(comb_hbm.at[base + ql], cbs[j],
                                  css[j]).wait()
            for c in range(N_CORNER):
                pltpu.make_async_copy(table_hbm.at[cbs[j].at[c]],
                                      gbs[j].at[pl.ds(c * 128, 128)],
                                      gss[j]).start()

        def compute(ql, j):
            for c in range(N_CORNER):
                pltpu.make_async_copy(table_hbm.at[cbs[j].at[c]],
                                      gbs[j].at[pl.ds(c * 128, 128)],
                                      gss[j]).wait()

            @pl.when(ql >= nbuf)
            def _():
                pltpu.make_async_copy(
                    obs[j], out_hbm.at[pl.ds((base + ql) * 2, 2)],
                    oss[j]).wait()

            @pl.loop(0, N_HEAD)
            def _(m):
                acc_e = jnp.zeros((16,), jnp.float32)
                acc_o = jnp.zeros((16,), jnp.float32)
                for c in range(N_CORNER):
                    for t in range(16):
                        off2 = m * 16 + t
                        wi = plsc.load_gather(
                            cbs[j], [jnp.full((16,), 4 + c, jnp.int32),
                                     jnp.full((16,), off2, jnp.int32)])
                        wf = plsc.bitcast(wi, jnp.float32)
                        rows = jnp.full((16,), c * 128 + off2, jnp.int32)
                        gi = plsc.load_gather(gbs[j], [rows, iota])
                        ge = plsc.bitcast(jax.lax.shift_left(gi, 16),
                                          jnp.float32)
                        go = plsc.bitcast(gi, jnp.float32)
                        acc_e = acc_e + wf * ge
                        acc_o = acc_o + wf * go
                orow0 = jnp.full((16,), m // 4, jnp.int32)
                ocol = (m % 4) * 32 + iota
                plsc.store_scatter(obs[j], [orow0, ocol], acc_e)
                plsc.store_scatter(obs[j], [orow0, ocol + 16], acc_o)

            pltpu.make_async_copy(
                obs[j], out_hbm.at[pl.ds((base + ql) * 2, 2)],
                oss[j]).start()

        for j in range(nbuf):
            startc(j, j)
        g_issue(0, 0)
        g_issue(1, 1)

        @pl.loop(0, QPW // nbuf)
        def _(kk):
            q0 = kk * nbuf
            for j in range(nbuf):
                ql = q0 + j
                compute(ql, j)

                @pl.when(ql + 2 < QPW)
                def _():
                    g_issue(ql + 2, (j + 2) % nbuf)

                @pl.when(ql + 4 < QPW)
                def _():
                    startc(ql + 4, j)

        for ql in range(QPW - nbuf, QPW):
            j = ql % nbuf
            pltpu.make_async_copy(
                obs[j], out_hbm.at[pl.ds((base + ql) * 2, 2)],
                oss[j]).wait()

    return k(table, comb)


def _layer_norm(x, g, b):
    m = jnp.mean(x, axis=-1, keepdims=True)
    v = jnp.mean((x - m) ** 2, axis=-1, keepdims=True)
    return (x - m) * jax.lax.rsqrt(v + 1e-5) * g + b


def _post_body(attn_ref, tgt_ref, wop_ref, bop_ref, g1_ref, b1_ref,
               wfc_ref, bfc_ref, wpr_ref, bpr_ref, g2_ref, b2_ref, out_ref):
    tgt2 = jnp.dot(attn_ref[0], wop_ref[...],
                   preferred_element_type=jnp.float32) + bop_ref[...]
    x = _layer_norm(tgt_ref[0] + tgt2, g1_ref[...], b1_ref[...])
    h = jnp.maximum(
        jnp.dot(x, wfc_ref[...], preferred_element_type=jnp.float32)
        + bfc_ref[...], 0.0)
    h = jnp.dot(h, wpr_ref[...], preferred_element_type=jnp.float32) + bpr_ref[...]
    out_ref[0] = _layer_norm(x + h, g2_ref[...], b2_ref[...])


def _post(attn, tgt_p, W_op, b_op, g1, b1, W_fc, b_fc, W_pr, b_pr, g2, b2):
    qspec = pl.BlockSpec((1, LQP, D_MODEL), lambda b: (b, 0, 0))
    return pl.pallas_call(
        _post_body,
        grid=(B,),
        in_specs=[
            qspec, qspec,
            pl.BlockSpec((D_MODEL, D_MODEL), lambda b: (0, 0)),
            pl.BlockSpec((1, D_MODEL), lambda b: (0, 0)),
            pl.BlockSpec((1, D_MODEL), lambda b: (0, 0)),
            pl.BlockSpec((1, D_MODEL), lambda b: (0, 0)),
            pl.BlockSpec((D_MODEL, 4 * D_MODEL), lambda b: (0, 0)),
            pl.BlockSpec((1, 4 * D_MODEL), lambda b: (0, 0)),
            pl.BlockSpec((4 * D_MODEL, D_MODEL), lambda b: (0, 0)),
            pl.BlockSpec((1, D_MODEL), lambda b: (0, 0)),
            pl.BlockSpec((1, D_MODEL), lambda b: (0, 0)),
            pl.BlockSpec((1, D_MODEL), lambda b: (0, 0)),
        ],
        out_specs=qspec,
        out_shape=jax.ShapeDtypeStruct((B, LQP, D_MODEL), jnp.float32),
    )(attn, tgt_p, W_op, b_op, g1, b1, W_fc, b_fc, W_pr, b_pr, g2, b2)


def kernel(tgt, query_pos, reference_points, src, src_spatial_shapes,
           level_start_index, src_padding_mask, W_so, b_so, W_aw, b_aw,
           W_vp, b_vp, W_op, b_op, ln1_g, ln1_b, W_fc, b_fc, W_pr, b_pr,
           ln2_g, ln2_b):
    maskf = (1.0 - src_padding_mask.astype(jnp.float32))[..., None]
    permv = jnp.asarray(_permv)
    value = _value_proj(src, W_vp[:, permv].astype(jnp.bfloat16),
                        b_vp[permv].reshape(1, -1), maskf)
    table = value.reshape(B * LEN_PAD * N_HEAD, 16)

    pad_q = [(0, 0), (0, LQP - LQ), (0, 0)]
    tgt_p = jnp.pad(tgt, pad_q)
    qpos_p = jnp.pad(query_pos, pad_q)
    ref_p = jnp.pad(reference_points.reshape(B, LQ, 8), pad_q)
    W_so_p = W_so[:, jnp.asarray(_perm)]
    b_so_p = b_so[jnp.asarray(_perm)].reshape(1, -1)

    qout = _qproj(tgt_p, qpos_p, ref_p, W_so_p, b_so_p, W_aw,
                  b_aw.reshape(1, -1))
    comb = qout.reshape(NQ, 8, NCOL)

    attn = _sc_attend(table, comb).reshape(B, LQP, D_MODEL)
    out = _post(attn, tgt_p, W_op, b_op.reshape(1, -1),
                ln1_g.reshape(1, -1), ln1_b.reshape(1, -1),
                W_fc, b_fc.reshape(1, -1), W_pr, b_pr.reshape(1, -1),
                ln2_g.reshape(1, -1), ln2_b.reshape(1, -1))
    return out[:, :LQ, :]


# speedup vs baseline: 1148.0001x; 1.0231x over previous
import functools

import jax
import jax.numpy as jnp
import numpy as np
from jax.experimental import pallas as pl
from jax.experimental.pallas import tpu as pltpu
from jax.experimental.pallas import tpu_sc as plsc

D_MODEL = 256
N_HEAD = 8
N_LEVELS = 4
N_POINTS = 4
HEAD_DIM = 32
B = 4
LQ = 900
LQP = 928
SPATIAL_SHAPES = [(100, 100), (50, 50), (25, 25), (13, 13)]
LEVEL_START = [0, 10000, 12500, 13125]
LEN_IN = 13294
LEN_PAD = 13312
NCOL = 128
N_CORNER = 4
N_IDX = B * LQP * N_CORNER * NCOL

_m_of_col = np.arange(NCOL) // 16
_l_of_col = (np.arange(NCOL) // 4) % 4
_W_of_col = np.array([SPATIAL_SHAPES[l][1] for l in _l_of_col], np.float32)
_H_of_col = np.array([SPATIAL_SHAPES[l][0] for l in _l_of_col], np.float32)
_LS_of_col = np.array([LEVEL_START[l] for l in _l_of_col], np.int32)

WVEC = _W_of_col[None, :]
HVEC = _H_of_col[None, :]
WVEC_I = _W_of_col.astype(np.int32)[None, :]
LSVEC_I = _LS_of_col[None, :]
MVEC_I = _m_of_col.astype(np.int32)[None, :]

_SelX = np.zeros((8, NCOL), np.float32)
_SelY = np.zeros((8, NCOL), np.float32)
for c in range(NCOL):
    l = _l_of_col[c]
    _SelX[2 * l + 0, c] = SPATIAL_SHAPES[l][1]
    _SelY[2 * l + 1, c] = SPATIAL_SHAPES[l][0]

_GM = np.kron(np.eye(8, dtype=np.float32), np.ones((16, 16), np.float32))

_perm = np.empty(256, np.int64)
for c in range(NCOL):
    m, l, p = c // 16, (c // 4) % 4, c % 4
    _perm[c] = ((m * 4 + l) * 4 + p) * 2
    _perm[128 + c] = ((m * 4 + l) * 4 + p) * 2 + 1

_permv = np.empty(256, np.int64)
for m in range(8):
    for k in range(16):
        _permv[m * 16 + k] = m * 32 + k
        _permv[128 + m * 16 + k] = m * 32 + 16 + k


_VROWS = 1024


def _value_proj_body(src_ref, w_ref, b_ref, mask_ref, out_ref):
    v = jnp.dot(src_ref[0].astype(jnp.bfloat16), w_ref[...],
                preferred_element_type=jnp.float32)
    v = (v + b_ref[...]) * mask_ref[0]

    def _rne(x):
        xb = jax.lax.bitcast_convert_type(x, jnp.int32)
        return xb + 0x7FFF + ((xb >> 16) & 1)

    out_ref[...] = jax.lax.shift_right_logical(_rne(v[:, :128]), 16) | (
        _rne(v[:, 128:]) & -65536)


def _value_proj(src, W_vp, b_vp, maskf):
    n_blk = LEN_PAD // _VROWS
    return pl.pallas_call(
        _value_proj_body,
        grid=(B, n_blk),
        in_specs=[
            pl.BlockSpec((1, _VROWS, D_MODEL), lambda b, i: (b, i, 0)),
            pl.BlockSpec((D_MODEL, D_MODEL), lambda b, i: (0, 0)),
            pl.BlockSpec((1, D_MODEL), lambda b, i: (0, 0)),
            pl.BlockSpec((1, _VROWS, 1), lambda b, i: (b, i, 0)),
        ],
        out_specs=pl.BlockSpec((_VROWS, 128), lambda b, i: (b * n_blk + i, 0)),
        out_shape=jax.ShapeDtypeStruct((B * LEN_PAD, 128), jnp.int32),
    )(src, W_vp, b_vp, maskf)


def _qproj_body(tgt_ref, qpos_ref, ref_ref, wso_ref, bso_ref, waw_ref,
                baw_ref, gm_ref, sel_ref, cst_ref, *out_refs):
    b = pl.program_id(0)
    q = tgt_ref[0] + qpos_ref[0]
    q = jnp.concatenate([q, jnp.zeros((LQP - LQ, D_MODEL), q.dtype)], axis=0)
    so = jnp.dot(q, wso_ref[...], preferred_element_type=jnp.float32) + bso_ref[...]
    logits = jnp.dot(q, waw_ref[...], preferred_element_type=jnp.float32) + baw_ref[...]
    e = jnp.exp(logits - jnp.max(logits, axis=-1, keepdims=True))
    denom = jnp.dot(e, gm_ref[...], preferred_element_type=jnp.float32)
    aw = e / denom

    refp = jnp.concatenate(
        [ref_ref[0], jnp.zeros((LQP - LQ, 8), jnp.float32)], axis=0)
    x = jnp.dot(refp, sel_ref[:8],
                preferred_element_type=jnp.float32) + so[:, :NCOL] - 0.5
    y = jnp.dot(refp, sel_ref[8:],
                preferred_element_type=jnp.float32) + so[:, NCOL:] - 0.5
    x0 = jnp.floor(x)
    y0 = jnp.floor(y)
    fx = x - x0
    fy = y - y0
    wvec = cst_ref[0:1]
    hvec = cst_ref[1:2]
    wvec_i = wvec.astype(jnp.int32)
    lsm_i = (cst_ref[2:3] * N_HEAD + cst_ref[3:4]).astype(jnp.int32)
    base = b * (LEN_PAD * N_HEAD)
    out_ref = out_refs[0]
    for c, (dx, dy) in enumerate(((0, 0), (1, 0), (0, 1), (1, 1))):
        ix = x0 + dx
        iy = y0 + dy
        vx = ((ix >= 0.0) & (ix <= wvec - 1.0)).astype(jnp.float32)
        vy = ((iy >= 0.0) & (iy <= hvec - 1.0)).astype(jnp.float32)
        ixc = jnp.clip(ix, 0.0, wvec - 1.0).astype(jnp.int32)
        iyc = jnp.clip(iy, 0.0, hvec - 1.0).astype(jnp.int32)
        wx = fx if dx else 1.0 - fx
        wy = fy if dy else 1.0 - fy
        sp = iyc * wvec_i + ixc
        out_ref[0, :, c, :] = base + sp * N_HEAD + lsm_i
        out_ref[0, :, 4 + c, :] = jax.lax.bitcast_convert_type(
            wx * wy * vx * vy * aw, jnp.int32)


def _qproj(tgt_p, qpos_p, ref_p, W_so_p, b_so_p, W_aw, b_aw):
    qspec = pl.BlockSpec((1, LQ, D_MODEL), lambda b: (b, 0, 0))
    ospec = pl.BlockSpec((1, LQP, 8, NCOL), lambda b: (b, 0, 0, 0))
    oshape = jax.ShapeDtypeStruct((B, LQP, 8, NCOL), jnp.int32)
    gm = jnp.asarray(_GM)
    sel = jnp.asarray(np.concatenate([_SelX, _SelY], axis=0))
    cst = jnp.asarray(np.stack([_W_of_col, _H_of_col,
                                _LS_of_col.astype(np.float32),
                                _m_of_col.astype(np.float32)], axis=0))
    return pl.pallas_call(
        _qproj_body,
        grid=(B,),
        in_specs=[
            qspec, qspec,
            pl.BlockSpec((1, LQ, 8), lambda b: (b, 0, 0)),
            pl.BlockSpec((D_MODEL, 256), lambda b: (0, 0)),
            pl.BlockSpec((1, 256), lambda b: (0, 0)),
            pl.BlockSpec((D_MODEL, NCOL), lambda b: (0, 0)),
            pl.BlockSpec((1, NCOL), lambda b: (0, 0)),
            pl.BlockSpec((NCOL, NCOL), lambda b: (0, 0)),
            pl.BlockSpec((16, NCOL), lambda b: (0, 0)),
            pl.BlockSpec((4, NCOL), lambda b: (0, 0)),
        ],
        out_specs=ospec,
        out_shape=oshape,
    )(tgt_p, qpos_p, ref_p, W_so_p, b_so_p, W_aw, b_aw, gm, sel, cst)


NW = 32
NQ = B * LQP
QPW = NQ // NW
NT = N_CORNER * NCOL


def _sc_attend(table, comb):
    mesh = plsc.VectorSubcoreMesh(core_axis_name="c", subcore_axis_name="s")

    nbuf = 4

    @functools.partial(
        pl.kernel,
        out_type=jax.ShapeDtypeStruct((NQ * 2, 128), jnp.float32),
        mesh=mesh,
        compiler_params=pltpu.CompilerParams(use_tc_tiling_on_sc=False,
                                             needs_layout_passes=False),
        scratch_types=(
            [pltpu.VMEM((8, 128), jnp.int32)] * nbuf
            + [pltpu.VMEM((NT, 16), jnp.int32)] * nbuf
            + [pltpu.VMEM((2, 128), jnp.float32)] * nbuf
            + [pltpu.SemaphoreType.DMA] * (3 * nbuf)
        ),
    )
    def k(table_hbm, comb_hbm, out_hbm, *scr):
        cbs, gbs, obs = scr[0:4], scr[4:8], scr[8:12]
        css, gss, oss = scr[12:16], scr[16:20], scr[20:24]
        iota = jax.lax.iota(jnp.int32, 16)
        wid = jax.lax.axis_index("s") * 2 + jax.lax.axis_index("c")
        base = wid * QPW

        def startc(ql, j):
            pltpu.make_async_copy(comb_hbm.at[base + ql], cbs[j],
                                  css[j]).start()

        def g_issue(ql, j):
            pltpu.make_async_copy(comb_hbm.at[base + ql], cbs[j],
                                  css[j]).wait()
            for c in range(N_CORNER):
                pltpu.make_async_copy(table_hbm.at[cbs[j].at[c]],
                                      gbs[j].at[pl.ds(c * 128, 128)],
                                      gss[j]).start()

        def compute(ql, j):
            for c in range(N_CORNER):
                pltpu.make_async_copy(table_hbm.at[cbs[j].at[c]],
                                      gbs[j].at[pl.ds(c * 128, 128)],
                                      gss[j]).wait()

            @pl.when(ql >= nbuf)
            def _():
                pltpu.make_async_copy(
                    obs[j], out_hbm.at[pl.ds((base + ql) * 2, 2)],
                    oss[j]).wait()

            @pl.loop(0, N_HEAD)
            def _(m):
                acc_e = jnp.zeros((16,), jnp.float32)
                acc_o = jnp.zeros((16,), jnp.float32)
                for c in range(N_CORNER):
                    for t in range(16):
                        off2 = m * 16 + t
                        wi = plsc.load_gather(
                            cbs[j], [jnp.full((16,), 4 + c, jnp.int32),
                                     jnp.full((16,), off2, jnp.int32)])
                        wf = plsc.bitcast(wi, jnp.float32)
                        rows = jnp.full((16,), c * 128 + off2, jnp.int32)
                        gi = plsc.load_gather(gbs[j], [rows, iota])
                        ge = plsc.bitcast(jax.lax.shift_left(gi, 16),
                                          jnp.float32)
                        go = plsc.bitcast(gi, jnp.float32)
                        acc_e = acc_e + wf * ge
                        acc_o = acc_o + wf * go
                orow0 = jnp.full((16,), m // 4, jnp.int32)
                ocol = (m % 4) * 32 + iota
                plsc.store_scatter(obs[j], [orow0, ocol], acc_e)
                plsc.store_scatter(obs[j], [orow0, ocol + 16], acc_o)

            pltpu.make_async_copy(
                obs[j], out_hbm.at[pl.ds((base + ql) * 2, 2)],
                oss[j]).start()

        for j in range(nbuf):
            startc(j, j)
        g_issue(0, 0)
        g_issue(1, 1)

        @pl.loop(0, QPW // nbuf)
        def _(kk):
            q0 = kk * nbuf
            for j in range(nbuf):
                ql = q0 + j
                compute(ql, j)

                @pl.when(ql + 2 < QPW)
                def _():
                    g_issue(ql + 2, (j + 2) % nbuf)

                @pl.when(ql + 4 < QPW)
                def _():
                    startc(ql + 4, j)

        for ql in range(QPW - nbuf, QPW):
            j = ql % nbuf
            pltpu.make_async_copy(
                obs[j], out_hbm.at[pl.ds((base + ql) * 2, 2)],
                oss[j]).wait()

    return k(table, comb)


def _layer_norm(x, g, b):
    m = jnp.mean(x, axis=-1, keepdims=True)
    v = jnp.mean((x - m) ** 2, axis=-1, keepdims=True)
    return (x - m) * jax.lax.rsqrt(v + 1e-5) * g + b


def _post_body(attn_ref, tgt_ref, wop_ref, bop_ref, g1_ref, b1_ref,
               wfc_ref, bfc_ref, wpr_ref, bpr_ref, g2_ref, b2_ref, out_ref):
    tgt2 = jnp.dot(attn_ref[0], wop_ref[...],
                   preferred_element_type=jnp.float32) + bop_ref[...]
    tgt_pad = jnp.concatenate(
        [tgt_ref[0], jnp.zeros((LQP - LQ, D_MODEL), jnp.float32)], axis=0)
    x = _layer_norm(tgt_pad + tgt2, g1_ref[...], b1_ref[...])
    h = jnp.maximum(
        jnp.dot(x, wfc_ref[...], preferred_element_type=jnp.float32)
        + bfc_ref[...], 0.0)
    h = jnp.dot(h, wpr_ref[...], preferred_element_type=jnp.float32) + bpr_ref[...]
    out_ref[0] = _layer_norm(x + h, g2_ref[...], b2_ref[...])


def _post(attn, tgt_p, W_op, b_op, g1, b1, W_fc, b_fc, W_pr, b_pr, g2, b2):
    qspec = pl.BlockSpec((1, LQP, D_MODEL), lambda b: (b, 0, 0))
    return pl.pallas_call(
        _post_body,
        grid=(B,),
        in_specs=[
            qspec, pl.BlockSpec((1, LQ, D_MODEL), lambda b: (b, 0, 0)),
            pl.BlockSpec((D_MODEL, D_MODEL), lambda b: (0, 0)),
            pl.BlockSpec((1, D_MODEL), lambda b: (0, 0)),
            pl.BlockSpec((1, D_MODEL), lambda b: (0, 0)),
            pl.BlockSpec((1, D_MODEL), lambda b: (0, 0)),
            pl.BlockSpec((D_MODEL, 4 * D_MODEL), lambda b: (0, 0)),
            pl.BlockSpec((1, 4 * D_MODEL), lambda b: (0, 0)),
            pl.BlockSpec((4 * D_MODEL, D_MODEL), lambda b: (0, 0)),
            pl.BlockSpec((1, D_MODEL), lambda b: (0, 0)),
            pl.BlockSpec((1, D_MODEL), lambda b: (0, 0)),
            pl.BlockSpec((1, D_MODEL), lambda b: (0, 0)),
        ],
        out_specs=qspec,
        out_shape=jax.ShapeDtypeStruct((B, LQP, D_MODEL), jnp.float32),
    )(attn, tgt_p, W_op, b_op, g1, b1, W_fc, b_fc, W_pr, b_pr, g2, b2)


def kernel(tgt, query_pos, reference_points, src, src_spatial_shapes,
           level_start_index, src_padding_mask, W_so, b_so, W_aw, b_aw,
           W_vp, b_vp, W_op, b_op, ln1_g, ln1_b, W_fc, b_fc, W_pr, b_pr,
           ln2_g, ln2_b):
    maskf = (1.0 - src_padding_mask.astype(jnp.float32))[..., None]
    permv = jnp.asarray(_permv)
    value = _value_proj(src, W_vp[:, permv].astype(jnp.bfloat16),
                        b_vp[permv].reshape(1, -1), maskf)
    table = value.reshape(B * LEN_PAD * N_HEAD, 16)

    W_so_p = W_so[:, jnp.asarray(_perm)]
    b_so_p = b_so[jnp.asarray(_perm)].reshape(1, -1)

    qout = _qproj(tgt, query_pos, reference_points.reshape(B, LQ, 8),
                  W_so_p, b_so_p, W_aw, b_aw.reshape(1, -1))
    comb = qout.reshape(NQ, 8, NCOL)

    attn = _sc_attend(table, comb).reshape(B, LQP, D_MODEL)
    out = _post(attn, tgt, W_op, b_op.reshape(1, -1),
                ln1_g.reshape(1, -1), ln1_b.reshape(1, -1),
                W_fc, b_fc.reshape(1, -1), W_pr, b_pr.reshape(1, -1),
                ln2_g.reshape(1, -1), ln2_b.reshape(1, -1))
    return out[:, :LQ, :]
